# baseline (speedup 1.0000x reference)
import functools

import jax
import jax.numpy as jnp
from jax import lax
from jax.experimental import pallas as pl
from jax.experimental.pallas import tpu as pltpu
from jax.experimental.pallas import tpu_sc as plsc

EMBED = 1024
HEADS = 16
HDIM = EMBED // HEADS
TOPK = 64


def _matmul_bias_kernel(x_ref, w_ref, b_ref, o_ref):
    o_ref[...] = (
        jnp.dot(x_ref[...], w_ref[...], preferred_element_type=jnp.float32)
        + b_ref[...]
    )


def _matmul_bias(x, w, b, block_m=512):
    m, kdim = x.shape
    n = w.shape[1]
    return pl.pallas_call(
        _matmul_bias_kernel,
        grid=(m // block_m,),
        in_specs=[
            pl.BlockSpec((block_m, kdim), lambda i: (i, 0)),
            pl.BlockSpec((kdim, n), lambda i: (0, 0)),
            pl.BlockSpec((n,), lambda i: (0,)),
        ],
        out_specs=pl.BlockSpec((block_m, n), lambda i: (i, 0)),
        out_shape=jax.ShapeDtypeStruct((m, n), jnp.float32),
    )(x, w, b)


def _proj_heads_kernel(x_ref, w_ref, b_ref, o_ref):
    o_ref[0] = (
        jnp.dot(x_ref[...], w_ref[0], preferred_element_type=jnp.float32)
        + b_ref[0]
    )


def _proj_heads(x, w, b, block_m=512):
    m, D = x.shape
    H, DH = HEADS, HDIM
    w3 = w.reshape(D, H, DH).transpose(1, 0, 2)
    b3 = b.reshape(H, 1, DH)
    return pl.pallas_call(
        _proj_heads_kernel,
        grid=(m // block_m, H),
        in_specs=[
            pl.BlockSpec((block_m, D), lambda i, h: (i, 0)),
            pl.BlockSpec((1, D, DH), lambda i, h: (h, 0, 0)),
            pl.BlockSpec((1, 1, DH), lambda i, h: (h, 0, 0)),
        ],
        out_specs=pl.BlockSpec((1, block_m, DH), lambda i, h: (h, i, 0)),
        out_shape=jax.ShapeDtypeStruct((H, m, DH), jnp.float32),
    )(x, w3, b3)


_NCELL = 256


def _scores_kernel(q_ref, k_ref, o_ref, g_ref, *, scale):
    s = (
        lax.dot_general(
            q_ref[0],
            k_ref[0],
            (((1,), (1,)), ((), ())),
            preferred_element_type=jnp.float32,
        )
        * scale
    )
    o_ref[0] = s
    bq = s.shape[0]
    g_ref[0] = jnp.max(s.reshape(bq, _NCELL, -1), axis=-1)


def _scores(q3, k3, B, LQ, LKV, block_q=256):
    H, DH = HEADS, HDIM
    grid = (B, H, LQ // block_q)
    return pl.pallas_call(
        functools.partial(_scores_kernel, scale=1.0 / (DH**0.5)),
        grid=grid,
        in_specs=[
            pl.BlockSpec(
                (1, block_q, DH),
                lambda b, h, i: (h, b * (LQ // block_q) + i, 0),
            ),
            pl.BlockSpec((1, LKV, DH), lambda b, h, i: (h, b, 0)),
        ],
        out_specs=[
            pl.BlockSpec(
                (1, block_q, LKV), lambda b, h, i: (b * HEADS + h, i, 0)
            ),
            pl.BlockSpec(
                (1, block_q, _NCELL), lambda b, h, i: (b * HEADS + h, i, 0)
            ),
        ],
        out_shape=[
            jax.ShapeDtypeStruct((B * H, LQ, LKV), jnp.float32),
            jax.ShapeDtypeStruct((B * H, LQ, _NCELL), jnp.float32),
        ],
    )(q3, k3)


def _vsort_d(x):
    s, _ = plsc.sort_key_val(x, x, descending=True)
    return s


def _rev(x):
    return lax.rev(x, dimensions=(0,))


def _sort64_desc(b0, b1, b2, b3):
    b0, b1, b2, b3 = _vsort_d(b0), _vsort_d(b1), _vsort_d(b2), _vsort_d(b3)

    def merge16(a, c):
        rc = _rev(c)
        return _vsort_d(jnp.maximum(a, rc)), _vsort_d(jnp.minimum(a, rc))

    x0, x1 = merge16(b0, b1)
    y0, y1 = merge16(b2, b3)
    ry0, ry1 = _rev(y1), _rev(y0)
    hi0, hi1 = jnp.maximum(x0, ry0), jnp.maximum(x1, ry1)
    lo0, lo1 = jnp.minimum(x0, ry0), jnp.minimum(x1, ry1)

    def clean32(p0, p1):
        return (
            _vsort_d(jnp.maximum(p0, p1)),
            _vsort_d(jnp.minimum(p0, p1)),
        )

    r0, r1 = clean32(hi0, hi1)
    r2, r3 = clean32(lo0, lo1)
    return r0, r1, r2, r3


def _merge_top64(r0, r1, r2, r3, s0, s1, s2, s3):
    m0 = jnp.maximum(r0, _rev(s3))
    m1 = jnp.maximum(r1, _rev(s2))
    m2 = jnp.maximum(r2, _rev(s1))
    m3 = jnp.maximum(r3, _rev(s0))
    p0, p1 = jnp.maximum(m0, m2), jnp.maximum(m1, m3)
    q0, q1 = jnp.minimum(m0, m2), jnp.minimum(m1, m3)

    def clean32(a, b):
        return (
            _vsort_d(jnp.maximum(a, b)),
            _vsort_d(jnp.minimum(a, b)),
        )

    r0, r1 = clean32(p0, p1)
    r2, r3 = clean32(q0, q1)
    return r0, r1, r2, r3


_SC_CHUNK = 8
_GROUP = 128


def _topk_sc(scores, gmax):
    R, LKV = scores.shape
    n_gmax = gmax.shape[1]
    mesh = plsc.VectorSubcoreMesh(core_axis_name="c", subcore_axis_name="s")
    info = plsc.get_sparse_core_info()
    n_workers = info.num_cores * info.num_subcores
    rows_per_worker = R // n_workers
    n_chunks = rows_per_worker // _SC_CHUNK

    @functools.partial(
        pl.kernel,
        mesh=mesh,
        out_type=jax.ShapeDtypeStruct((R, TOPK), jnp.float32),
        scratch_types=[
            pltpu.VMEM((_SC_CHUNK, LKV), jnp.float32),
            pltpu.VMEM((_SC_CHUNK, TOPK), jnp.float32),
            pltpu.VMEM((_SC_CHUNK, n_gmax), jnp.float32),
            pltpu.VMEM((_SC_CHUNK, LKV + TOPK), jnp.float32),
        ],
        compiler_params=pltpu.CompilerParams(needs_layout_passes=False),
    )
    def topk_kernel(scores_hbm, gmax_hbm, out_hbm, rows_v, out_v, gmax_v, cand_v):
        wid = lax.axis_index("s") * info.num_cores + lax.axis_index("c")
        base = wid * rows_per_worker

        def chunk_body(c, _):
            row0 = base + c * _SC_CHUNK
            pltpu.sync_copy(scores_hbm.at[pl.ds(row0, _SC_CHUNK)], rows_v)
            pltpu.sync_copy(gmax_hbm.at[pl.ds(row0, _SC_CHUNK)], gmax_v)

            @plsc.parallel_loop(0, _SC_CHUNK, 1, unroll=2)
            def row_body(r):
                def ld64(ref, base_el):
                    return (
                        ref[r, pl.ds(base_el, 16)],
                        ref[r, pl.ds(base_el + 16, 16)],
                        ref[r, pl.ds(base_el + 32, 16)],
                        ref[r, pl.ds(base_el + 48, 16)],
                    )

                g0, g1, g2, g3 = ld64(gmax_v, 0)
                r0, r1, r2, r3 = _sort64_desc(g0, g1, g2, g3)

                def t_body(bi, carry):
                    r0, r1, r2, r3 = carry
                    b0, b1, b2, b3 = ld64(gmax_v, bi * 64)
                    s0, s1, s2, s3 = _sort64_desc(b0, b1, b2, b3)
                    return _merge_top64(r0, r1, r2, r3, s0, s1, s2, s3)

                r0, r1, r2, r3 = lax.fori_loop(
                    1, n_gmax // 64, t_body, (r0, r1, r2, r3)
                )
                t0 = jnp.min(r3)

                def c_body(j, cnt):
                    for u in range(8):
                        x = rows_v[r, pl.ds(j * 128 + u * 16, 16)]
                        msk = x >= t0
                        plsc.store_compressed(
                            cand_v.at[r, pl.ds(cnt, 16)], x, mask=msk
                        )
                        pc = plsc.all_reduce_population_count(msk)
                        cnt = cnt + pc[0]
                    return cnt

                cnt = lax.fori_loop(0, LKV // 128, c_body, 0)

                neg = jnp.full((16,), -jnp.inf, dtype=jnp.float32)
                cand_v[r, pl.ds(cnt, 16)] = neg
                cand_v[r, pl.ds(cnt + 16, 16)] = neg
                cand_v[r, pl.ds(cnt + 32, 16)] = neg
                cand_v[r, pl.ds(cnt + 48, 16)] = neg

                b0, b1, b2, b3 = ld64(cand_v, 0)
                r0, r1, r2, r3 = _sort64_desc(b0, b1, b2, b3)
                t = jnp.min(r3)
                nb = (cnt + 63) // 64

                def batch_body(bi, carry):
                    r0, r1, r2, r3, t = carry
                    b0, b1, b2, b3 = ld64(cand_v, bi * 64)
                    bmax = jnp.max(
                        jnp.maximum(jnp.maximum(b0, b1), jnp.maximum(b2, b3))
                    )

                    def do_merge(args):
                        r0, r1, r2, r3, b0, b1, b2, b3 = args
                        s0, s1, s2, s3 = _sort64_desc(b0, b1, b2, b3)
                        n0, n1, n2, n3 = _merge_top64(
                            r0, r1, r2, r3, s0, s1, s2, s3
                        )
                        return n0, n1, n2, n3, jnp.min(n3)

                    def no_merge(args):
                        r0, r1, r2, r3, b0, b1, b2, b3 = args
                        return r0, r1, r2, r3, t

                    return lax.cond(
                        bmax > t,
                        do_merge,
                        no_merge,
                        (r0, r1, r2, r3, b0, b1, b2, b3),
                    )

                r0, r1, r2, r3, t = lax.fori_loop(
                    1, nb, batch_body, (r0, r1, r2, r3, t)
                )
                out_v[r, pl.ds(0, 16)] = r0
                out_v[r, pl.ds(16, 16)] = r1
                out_v[r, pl.ds(32, 16)] = r2
                out_v[r, pl.ds(48, 16)] = r3

            pltpu.sync_copy(out_v, out_hbm.at[pl.ds(row0, _SC_CHUNK)])
            return ()

        lax.fori_loop(0, n_chunks, chunk_body, ())

    return topk_kernel(scores, gmax)


def _attend_kernel(s_ref, tk_ref, v_ref, att_ref, sm_ref):
    tv = tk_ref[0]
    m = tv[:, 0:1]
    e = jnp.exp(tv - m)
    denom = jnp.sum(e, axis=1, keepdims=True)
    thr = tv[:, TOPK - 1 :]
    sm_ref[0] = e / denom
    s = s_ref[0]
    p = jnp.where(s >= thr, jnp.exp(s - m), 0.0) / denom
    att_ref[0] = jnp.dot(p, v_ref[0], preferred_element_type=jnp.float32)


def _attend(scores, topk, v3, B, LQ, LKV, block_q=512):
    H, DH = HEADS, HDIM
    BH = B * H
    grid = (BH, LQ // block_q)
    return pl.pallas_call(
        _attend_kernel,
        grid=grid,
        in_specs=[
            pl.BlockSpec((1, block_q, LKV), lambda bh, i: (bh, i, 0)),
            pl.BlockSpec((1, block_q, TOPK), lambda bh, i: (bh, i, 0)),
            pl.BlockSpec(
                (1, LKV, DH), lambda bh, i: (bh % HEADS, bh // HEADS, 0)
            ),
        ],
        out_specs=[
            pl.BlockSpec((1, block_q, DH), lambda bh, i: (bh, i, 0)),
            pl.BlockSpec((1, block_q, TOPK), lambda bh, i: (bh, i, 0)),
        ],
        out_shape=[
            jax.ShapeDtypeStruct((BH, LQ, DH), jnp.float32),
            jax.ShapeDtypeStruct((BH, LQ, TOPK), jnp.float32),
        ],
    )(scores, topk, v3)


def _out_proj_kernel(a_ref, w_ref, b_ref, o_ref):
    h = pl.program_id(2)

    @pl.when(h == 0)
    def _():
        o_ref[...] = jnp.broadcast_to(b_ref[...], o_ref.shape)

    o_ref[...] += jnp.dot(
        a_ref[0, 0], w_ref[0], preferred_element_type=jnp.float32
    )


def _out_proj(att, Wo, bo, B, LQ, block_m=512):
    H, DH, D = HEADS, HDIM, EMBED
    att4 = att.reshape(B, H, LQ, DH)
    wo3 = Wo.reshape(H, DH, D)
    grid = (B, LQ // block_m, H)
    return pl.pallas_call(
        _out_proj_kernel,
        grid=grid,
        in_specs=[
            pl.BlockSpec((1, 1, block_m, DH), lambda b, i, h: (b, h, i, 0)),
            pl.BlockSpec((1, DH, D), lambda b, i, h: (h, 0, 0)),
            pl.BlockSpec((D,), lambda b, i, h: (0,)),
        ],
        out_specs=pl.BlockSpec(
            (1, block_m, D), lambda b, i, h: (b, i, 0)
        ),
        out_shape=jax.ShapeDtypeStruct((B, LQ, D), jnp.float32),
    )(att4, wo3, bo)


def kernel(local_feat, global_feat, Wq, bq, Wk, bk, Wv, bv, Wo, bo):
    B, LQ, D = local_feat.shape
    LKV = global_feat.shape[1]
    H = HEADS

    q3 = _proj_heads(local_feat.reshape(B * LQ, D), Wq, bq)
    k3 = _proj_heads(global_feat.reshape(B * LKV, D), Wk, bk)
    v3 = _proj_heads(global_feat.reshape(B * LKV, D), Wv, bv)

    scores, gmax = _scores(q3, k3, B, LQ, LKV)

    topk = _topk_sc(
        scores.reshape(B * H * LQ, LKV), gmax.reshape(B * H * LQ, _NCELL)
    ).reshape(B * H, LQ, TOPK)

    att, sm = _attend(scores, topk, v3, B, LQ, LKV)

    output = _out_proj(att, Wo, bo, B, LQ)
    return (output, sm.reshape(B, H, LQ, TOPK))


# speedup vs baseline: 2.0247x; 2.0247x over previous
import functools

import jax
import jax.numpy as jnp
from jax import lax
from jax.experimental import pallas as pl
from jax.experimental.pallas import tpu as pltpu
from jax.experimental.pallas import tpu_sc as plsc

EMBED = 1024
HEADS = 16
HDIM = EMBED // HEADS
TOPK = 64


def _matmul_bias_kernel(x_ref, w_ref, b_ref, o_ref):
    o_ref[...] = (
        jnp.dot(x_ref[...], w_ref[...], preferred_element_type=jnp.float32)
        + b_ref[...]
    )


def _matmul_bias(x, w, b, block_m=512):
    m, kdim = x.shape
    n = w.shape[1]
    return pl.pallas_call(
        _matmul_bias_kernel,
        grid=(m // block_m,),
        in_specs=[
            pl.BlockSpec((block_m, kdim), lambda i: (i, 0)),
            pl.BlockSpec((kdim, n), lambda i: (0, 0)),
            pl.BlockSpec((n,), lambda i: (0,)),
        ],
        out_specs=pl.BlockSpec((block_m, n), lambda i: (i, 0)),
        out_shape=jax.ShapeDtypeStruct((m, n), jnp.float32),
    )(x, w, b)


def _proj_heads_kernel(x_ref, w_ref, b_ref, o_ref):
    o_ref[0] = (
        jnp.dot(x_ref[...], w_ref[0], preferred_element_type=jnp.float32)
        + b_ref[0]
    )


def _proj_heads(x, w, b, block_m=512):
    m, D = x.shape
    H, DH = HEADS, HDIM
    w3 = w.reshape(D, H, DH).transpose(1, 0, 2)
    b3 = b.reshape(H, 1, DH)
    return pl.pallas_call(
        _proj_heads_kernel,
        grid=(m // block_m, H),
        in_specs=[
            pl.BlockSpec((block_m, D), lambda i, h: (i, 0)),
            pl.BlockSpec((1, D, DH), lambda i, h: (h, 0, 0)),
            pl.BlockSpec((1, 1, DH), lambda i, h: (h, 0, 0)),
        ],
        out_specs=pl.BlockSpec((1, block_m, DH), lambda i, h: (h, i, 0)),
        out_shape=jax.ShapeDtypeStruct((H, m, DH), jnp.float32),
    )(x, w3, b3)


_NCELL = 128


def _scores_kernel(q_ref, k_ref, o_ref, g_ref, *, scale):
    s = (
        lax.dot_general(
            q_ref[0],
            k_ref[0],
            (((1,), (1,)), ((), ())),
            preferred_element_type=jnp.float32,
        )
        * scale
    )
    o_ref[0] = s
    bq, lkv = s.shape
    gm = lax.slice(s, (0, 0), (bq, _NCELL))
    for k in range(1, lkv // _NCELL):
        gm = jnp.maximum(
            gm, lax.slice(s, (0, k * _NCELL), (bq, (k + 1) * _NCELL))
        )
    g_ref[0] = gm


def _scores(q3, k3, B, LQ, LKV, block_q=256):
    H, DH = HEADS, HDIM
    grid = (B, H, LQ // block_q)
    return pl.pallas_call(
        functools.partial(_scores_kernel, scale=1.0 / (DH**0.5)),
        grid=grid,
        in_specs=[
            pl.BlockSpec(
                (1, block_q, DH),
                lambda b, h, i: (h, b * (LQ // block_q) + i, 0),
            ),
            pl.BlockSpec((1, LKV, DH), lambda b, h, i: (h, b, 0)),
        ],
        out_specs=[
            pl.BlockSpec(
                (1, block_q, LKV), lambda b, h, i: (b * HEADS + h, i, 0)
            ),
            pl.BlockSpec(
                (1, block_q, _NCELL), lambda b, h, i: (b * HEADS + h, i, 0)
            ),
        ],
        out_shape=[
            jax.ShapeDtypeStruct((B * H, LQ, LKV), jnp.float32),
            jax.ShapeDtypeStruct((B * H, LQ, _NCELL), jnp.float32),
        ],
    )(q3, k3)


def _vsort_d(x):
    s, _ = plsc.sort_key_val(x, x, descending=True)
    return s


def _rev(x):
    return lax.rev(x, dimensions=(0,))


def _sort64_desc(b0, b1, b2, b3):
    b0, b1, b2, b3 = _vsort_d(b0), _vsort_d(b1), _vsort_d(b2), _vsort_d(b3)

    def merge16(a, c):
        rc = _rev(c)
        return _vsort_d(jnp.maximum(a, rc)), _vsort_d(jnp.minimum(a, rc))

    x0, x1 = merge16(b0, b1)
    y0, y1 = merge16(b2, b3)
    ry0, ry1 = _rev(y1), _rev(y0)
    hi0, hi1 = jnp.maximum(x0, ry0), jnp.maximum(x1, ry1)
    lo0, lo1 = jnp.minimum(x0, ry0), jnp.minimum(x1, ry1)

    def clean32(p0, p1):
        return (
            _vsort_d(jnp.maximum(p0, p1)),
            _vsort_d(jnp.minimum(p0, p1)),
        )

    r0, r1 = clean32(hi0, hi1)
    r2, r3 = clean32(lo0, lo1)
    return r0, r1, r2, r3


def _merge_top64(r0, r1, r2, r3, s0, s1, s2, s3):
    m0 = jnp.maximum(r0, _rev(s3))
    m1 = jnp.maximum(r1, _rev(s2))
    m2 = jnp.maximum(r2, _rev(s1))
    m3 = jnp.maximum(r3, _rev(s0))
    p0, p1 = jnp.maximum(m0, m2), jnp.maximum(m1, m3)
    q0, q1 = jnp.minimum(m0, m2), jnp.minimum(m1, m3)

    def clean32(a, b):
        return (
            _vsort_d(jnp.maximum(a, b)),
            _vsort_d(jnp.minimum(a, b)),
        )

    r0, r1 = clean32(p0, p1)
    r2, r3 = clean32(q0, q1)
    return r0, r1, r2, r3


_SC_CHUNK = 8
_GROUP = 128


def _topk_sc(scores, gmax):
    R, LKV = scores.shape
    n_gmax = gmax.shape[1]
    mesh = plsc.VectorSubcoreMesh(core_axis_name="c", subcore_axis_name="s")
    info = plsc.get_sparse_core_info()
    n_workers = info.num_cores * info.num_subcores
    rows_per_worker = R // n_workers
    n_chunks = rows_per_worker // _SC_CHUNK

    @functools.partial(
        pl.kernel,
        mesh=mesh,
        out_type=jax.ShapeDtypeStruct((R, TOPK), jnp.float32),
        scratch_types=[
            pltpu.VMEM((_SC_CHUNK, LKV), jnp.float32),
            pltpu.VMEM((_SC_CHUNK, TOPK), jnp.float32),
            pltpu.VMEM((_SC_CHUNK, n_gmax), jnp.float32),
            pltpu.VMEM((_SC_CHUNK, LKV + TOPK), jnp.float32),
        ],
        compiler_params=pltpu.CompilerParams(needs_layout_passes=False),
    )
    def topk_kernel(scores_hbm, gmax_hbm, out_hbm, rows_v, out_v, gmax_v, cand_v):
        wid = lax.axis_index("s") * info.num_cores + lax.axis_index("c")
        base = wid * rows_per_worker

        def chunk_body(c, _):
            row0 = base + c * _SC_CHUNK
            pltpu.sync_copy(scores_hbm.at[pl.ds(row0, _SC_CHUNK)], rows_v)
            pltpu.sync_copy(gmax_hbm.at[pl.ds(row0, _SC_CHUNK)], gmax_v)

            @plsc.parallel_loop(0, _SC_CHUNK, 1, unroll=2)
            def row_body(r):
                def ld64(ref, base_el):
                    return (
                        ref[r, pl.ds(base_el, 16)],
                        ref[r, pl.ds(base_el + 16, 16)],
                        ref[r, pl.ds(base_el + 32, 16)],
                        ref[r, pl.ds(base_el + 48, 16)],
                    )

                g0, g1, g2, g3 = ld64(gmax_v, 0)
                r0, r1, r2, r3 = _sort64_desc(g0, g1, g2, g3)

                def t_body(bi, carry):
                    r0, r1, r2, r3 = carry
                    b0, b1, b2, b3 = ld64(gmax_v, bi * 64)
                    s0, s1, s2, s3 = _sort64_desc(b0, b1, b2, b3)
                    return _merge_top64(r0, r1, r2, r3, s0, s1, s2, s3)

                r0, r1, r2, r3 = lax.fori_loop(
                    1, n_gmax // 64, t_body, (r0, r1, r2, r3)
                )
                t0 = jnp.min(r3)

                def c_body(j, cnt):
                    for u in range(8):
                        x = rows_v[r, pl.ds(j * 128 + u * 16, 16)]
                        msk = x >= t0
                        plsc.store_compressed(
                            cand_v.at[r, pl.ds(cnt, 16)], x, mask=msk
                        )
                        pc = plsc.all_reduce_population_count(msk)
                        cnt = cnt + pc[0]
                    return cnt

                cnt = lax.fori_loop(0, LKV // 128, c_body, 0)

                neg = jnp.full((16,), -jnp.inf, dtype=jnp.float32)
                cand_v[r, pl.ds(cnt, 16)] = neg
                cand_v[r, pl.ds(cnt + 16, 16)] = neg
                cand_v[r, pl.ds(cnt + 32, 16)] = neg
                cand_v[r, pl.ds(cnt + 48, 16)] = neg

                b0, b1, b2, b3 = ld64(cand_v, 0)
                r0, r1, r2, r3 = _sort64_desc(b0, b1, b2, b3)
                t = jnp.min(r3)
                nb = (cnt + 63) // 64

                def batch_body(bi, carry):
                    r0, r1, r2, r3, t = carry
                    b0, b1, b2, b3 = ld64(cand_v, bi * 64)
                    bmax = jnp.max(
                        jnp.maximum(jnp.maximum(b0, b1), jnp.maximum(b2, b3))
                    )

                    def do_merge(args):
                        r0, r1, r2, r3, b0, b1, b2, b3 = args
                        s0, s1, s2, s3 = _sort64_desc(b0, b1, b2, b3)
                        n0, n1, n2, n3 = _merge_top64(
                            r0, r1, r2, r3, s0, s1, s2, s3
                        )
                        return n0, n1, n2, n3, jnp.min(n3)

                    def no_merge(args):
                        r0, r1, r2, r3, b0, b1, b2, b3 = args
                        return r0, r1, r2, r3, t

                    return lax.cond(
                        bmax > t,
                        do_merge,
                        no_merge,
                        (r0, r1, r2, r3, b0, b1, b2, b3),
                    )

                r0, r1, r2, r3, t = lax.fori_loop(
                    1, nb, batch_body, (r0, r1, r2, r3, t)
                )
                out_v[r, pl.ds(0, 16)] = r0
                out_v[r, pl.ds(16, 16)] = r1
                out_v[r, pl.ds(32, 16)] = r2
                out_v[r, pl.ds(48, 16)] = r3

            pltpu.sync_copy(out_v, out_hbm.at[pl.ds(row0, _SC_CHUNK)])
            return ()

        lax.fori_loop(0, n_chunks, chunk_body, ())

    return topk_kernel(scores, gmax)


def _attend_kernel(s_ref, tk_ref, v_ref, att_ref, sm_ref):
    tv = tk_ref[0]
    m = tv[:, 0:1]
    e = jnp.exp(tv - m)
    denom = jnp.sum(e, axis=1, keepdims=True)
    thr = tv[:, TOPK - 1 :]
    sm_ref[0] = e / denom
    s = s_ref[0]
    p = jnp.where(s >= thr, jnp.exp(s - m), 0.0) / denom
    att_ref[0] = jnp.dot(p, v_ref[0], preferred_element_type=jnp.float32)


def _attend(scores, topk, v3, B, LQ, LKV, block_q=512):
    H, DH = HEADS, HDIM
    BH = B * H
    grid = (BH, LQ // block_q)
    return pl.pallas_call(
        _attend_kernel,
        grid=grid,
        in_specs=[
            pl.BlockSpec((1, block_q, LKV), lambda bh, i: (bh, i, 0)),
            pl.BlockSpec((1, block_q, TOPK), lambda bh, i: (bh, i, 0)),
            pl.BlockSpec(
                (1, LKV, DH), lambda bh, i: (bh % HEADS, bh // HEADS, 0)
            ),
        ],
        out_specs=[
            pl.BlockSpec((1, block_q, DH), lambda bh, i: (bh, i, 0)),
            pl.BlockSpec((1, block_q, TOPK), lambda bh, i: (bh, i, 0)),
        ],
        out_shape=[
            jax.ShapeDtypeStruct((BH, LQ, DH), jnp.float32),
            jax.ShapeDtypeStruct((BH, LQ, TOPK), jnp.float32),
        ],
    )(scores, topk, v3)


def _out_proj_kernel(a_ref, w_ref, b_ref, o_ref):
    h = pl.program_id(2)

    @pl.when(h == 0)
    def _():
        o_ref[...] = jnp.broadcast_to(b_ref[...], o_ref.shape)

    o_ref[...] += jnp.dot(
        a_ref[0, 0], w_ref[0], preferred_element_type=jnp.float32
    )


def _out_proj(att, Wo, bo, B, LQ, block_m=512):
    H, DH, D = HEADS, HDIM, EMBED
    att4 = att.reshape(B, H, LQ, DH)
    wo3 = Wo.reshape(H, DH, D)
    grid = (B, LQ // block_m, H)
    return pl.pallas_call(
        _out_proj_kernel,
        grid=grid,
        in_specs=[
            pl.BlockSpec((1, 1, block_m, DH), lambda b, i, h: (b, h, i, 0)),
            pl.BlockSpec((1, DH, D), lambda b, i, h: (h, 0, 0)),
            pl.BlockSpec((D,), lambda b, i, h: (0,)),
        ],
        out_specs=pl.BlockSpec(
            (1, block_m, D), lambda b, i, h: (b, i, 0)
        ),
        out_shape=jax.ShapeDtypeStruct((B, LQ, D), jnp.float32),
    )(att4, wo3, bo)


def kernel(local_feat, global_feat, Wq, bq, Wk, bk, Wv, bv, Wo, bo):
    B, LQ, D = local_feat.shape
    LKV = global_feat.shape[1]
    H = HEADS

    q3 = _proj_heads(local_feat.reshape(B * LQ, D), Wq, bq)
    k3 = _proj_heads(global_feat.reshape(B * LKV, D), Wk, bk)
    v3 = _proj_heads(global_feat.reshape(B * LKV, D), Wv, bv)

    scores, gmax = _scores(q3, k3, B, LQ, LKV)

    topk = _topk_sc(
        scores.reshape(B * H * LQ, LKV), gmax.reshape(B * H * LQ, _NCELL)
    ).reshape(B * H, LQ, TOPK)

    att, sm = _attend(scores, topk, v3, B, LQ, LKV)

    output = _out_proj(att, Wo, bo, B, LQ)
    return (output, sm.reshape(B, H, LQ, TOPK))


# speedup vs baseline: 2.0539x; 1.0144x over previous
import functools

import jax
import jax.numpy as jnp
from jax import lax
from jax.experimental import pallas as pl
from jax.experimental.pallas import tpu as pltpu
from jax.experimental.pallas import tpu_sc as plsc

EMBED = 1024
HEADS = 16
HDIM = EMBED // HEADS
TOPK = 64


def _matmul_bias_kernel(x_ref, w_ref, b_ref, o_ref):
    o_ref[...] = (
        jnp.dot(x_ref[...], w_ref[...], preferred_element_type=jnp.float32)
        + b_ref[...]
    )


def _matmul_bias(x, w, b, block_m=512):
    m, kdim = x.shape
    n = w.shape[1]
    return pl.pallas_call(
        _matmul_bias_kernel,
        grid=(m // block_m,),
        in_specs=[
            pl.BlockSpec((block_m, kdim), lambda i: (i, 0)),
            pl.BlockSpec((kdim, n), lambda i: (0, 0)),
            pl.BlockSpec((n,), lambda i: (0,)),
        ],
        out_specs=pl.BlockSpec((block_m, n), lambda i: (i, 0)),
        out_shape=jax.ShapeDtypeStruct((m, n), jnp.float32),
    )(x, w, b)


def _proj_heads_kernel(x_ref, w_ref, b_ref, o_ref):
    o_ref[0] = (
        jnp.dot(x_ref[...], w_ref[0], preferred_element_type=jnp.float32)
        + b_ref[0]
    )


def _proj_heads(x, w, b, block_m=512):
    m, D = x.shape
    H, DH = HEADS, HDIM
    w3 = w.reshape(D, H, DH).transpose(1, 0, 2)
    b3 = b.reshape(H, 1, DH)
    return pl.pallas_call(
        _proj_heads_kernel,
        grid=(m // block_m, H),
        in_specs=[
            pl.BlockSpec((block_m, D), lambda i, h: (i, 0)),
            pl.BlockSpec((1, D, DH), lambda i, h: (h, 0, 0)),
            pl.BlockSpec((1, 1, DH), lambda i, h: (h, 0, 0)),
        ],
        out_specs=pl.BlockSpec((1, block_m, DH), lambda i, h: (h, i, 0)),
        out_shape=jax.ShapeDtypeStruct((H, m, DH), jnp.float32),
    )(x, w3, b3)


_NCELL = 128


def _scores_kernel(q_ref, k_ref, o_ref, g_ref, *, scale):
    s = (
        lax.dot_general(
            q_ref[0],
            k_ref[0],
            (((1,), (1,)), ((), ())),
            preferred_element_type=jnp.float32,
        )
        * scale
    )
    o_ref[0] = s
    bq, lkv = s.shape
    gm = lax.slice(s, (0, 0), (bq, _NCELL))
    for k in range(1, lkv // _NCELL):
        gm = jnp.maximum(
            gm, lax.slice(s, (0, k * _NCELL), (bq, (k + 1) * _NCELL))
        )
    g_ref[0] = gm


def _scores(q3, k3, B, LQ, LKV, block_q=256):
    H, DH = HEADS, HDIM
    grid = (B, H, LQ // block_q)
    return pl.pallas_call(
        functools.partial(_scores_kernel, scale=1.0 / (DH**0.5)),
        grid=grid,
        in_specs=[
            pl.BlockSpec(
                (1, block_q, DH),
                lambda b, h, i: (h, b * (LQ // block_q) + i, 0),
            ),
            pl.BlockSpec((1, LKV, DH), lambda b, h, i: (h, b, 0)),
        ],
        out_specs=[
            pl.BlockSpec(
                (1, block_q, LKV), lambda b, h, i: (b * HEADS + h, i, 0)
            ),
            pl.BlockSpec(
                (1, block_q, _NCELL), lambda b, h, i: (b * HEADS + h, i, 0)
            ),
        ],
        out_shape=[
            jax.ShapeDtypeStruct((B * H, LQ, LKV), jnp.float32),
            jax.ShapeDtypeStruct((B * H, LQ, _NCELL), jnp.float32),
        ],
    )(q3, k3)


def _vsort_d(x):
    s, _ = plsc.sort_key_val(x, x, descending=True)
    return s


def _rev(x):
    return lax.rev(x, dimensions=(0,))


def _sort64_desc(b0, b1, b2, b3):
    b0, b1, b2, b3 = _vsort_d(b0), _vsort_d(b1), _vsort_d(b2), _vsort_d(b3)

    def merge16(a, c):
        rc = _rev(c)
        return _vsort_d(jnp.maximum(a, rc)), _vsort_d(jnp.minimum(a, rc))

    x0, x1 = merge16(b0, b1)
    y0, y1 = merge16(b2, b3)
    ry0, ry1 = _rev(y1), _rev(y0)
    hi0, hi1 = jnp.maximum(x0, ry0), jnp.maximum(x1, ry1)
    lo0, lo1 = jnp.minimum(x0, ry0), jnp.minimum(x1, ry1)

    def clean32(p0, p1):
        return (
            _vsort_d(jnp.maximum(p0, p1)),
            _vsort_d(jnp.minimum(p0, p1)),
        )

    r0, r1 = clean32(hi0, hi1)
    r2, r3 = clean32(lo0, lo1)
    return r0, r1, r2, r3


def _merge_top64(r0, r1, r2, r3, s0, s1, s2, s3):
    m0 = jnp.maximum(r0, _rev(s3))
    m1 = jnp.maximum(r1, _rev(s2))
    m2 = jnp.maximum(r2, _rev(s1))
    m3 = jnp.maximum(r3, _rev(s0))
    p0, p1 = jnp.maximum(m0, m2), jnp.maximum(m1, m3)
    q0, q1 = jnp.minimum(m0, m2), jnp.minimum(m1, m3)

    def clean32(a, b):
        return (
            _vsort_d(jnp.maximum(a, b)),
            _vsort_d(jnp.minimum(a, b)),
        )

    r0, r1 = clean32(p0, p1)
    r2, r3 = clean32(q0, q1)
    return r0, r1, r2, r3


_SC_CHUNK = 16
_GROUP = 128


def _topk_sc(scores, gmax):
    R, LKV = scores.shape
    n_gmax = gmax.shape[1]
    mesh = plsc.VectorSubcoreMesh(core_axis_name="c", subcore_axis_name="s")
    info = plsc.get_sparse_core_info()
    n_workers = info.num_cores * info.num_subcores
    rows_per_worker = R // n_workers
    n_chunks = rows_per_worker // _SC_CHUNK

    @functools.partial(
        pl.kernel,
        mesh=mesh,
        out_type=jax.ShapeDtypeStruct((R, TOPK), jnp.float32),
        scratch_types=[
            pltpu.VMEM((_SC_CHUNK, LKV), jnp.float32),
            pltpu.VMEM((_SC_CHUNK, TOPK), jnp.float32),
            pltpu.VMEM((_SC_CHUNK, n_gmax), jnp.float32),
            pltpu.VMEM((_SC_CHUNK, LKV + TOPK), jnp.float32),
        ],
        compiler_params=pltpu.CompilerParams(needs_layout_passes=False),
    )
    def topk_kernel(scores_hbm, gmax_hbm, out_hbm, rows_v, out_v, gmax_v, cand_v):
        wid = lax.axis_index("s") * info.num_cores + lax.axis_index("c")
        base = wid * rows_per_worker

        def chunk_body(c, _):
            row0 = base + c * _SC_CHUNK
            pltpu.sync_copy(scores_hbm.at[pl.ds(row0, _SC_CHUNK)], rows_v)
            pltpu.sync_copy(gmax_hbm.at[pl.ds(row0, _SC_CHUNK)], gmax_v)

            @plsc.parallel_loop(0, _SC_CHUNK, 1, unroll=2)
            def row_body(r):
                def ld64(ref, base_el):
                    return (
                        ref[r, pl.ds(base_el, 16)],
                        ref[r, pl.ds(base_el + 16, 16)],
                        ref[r, pl.ds(base_el + 32, 16)],
                        ref[r, pl.ds(base_el + 48, 16)],
                    )

                g0, g1, g2, g3 = ld64(gmax_v, 0)
                r0, r1, r2, r3 = _sort64_desc(g0, g1, g2, g3)

                def t_body(bi, carry):
                    r0, r1, r2, r3 = carry
                    b0, b1, b2, b3 = ld64(gmax_v, bi * 64)
                    s0, s1, s2, s3 = _sort64_desc(b0, b1, b2, b3)
                    return _merge_top64(r0, r1, r2, r3, s0, s1, s2, s3)

                r0, r1, r2, r3 = lax.fori_loop(
                    1, n_gmax // 64, t_body, (r0, r1, r2, r3)
                )
                t0 = jnp.min(r3)

                def c_body(j, cnt):
                    for u in range(8):
                        x = rows_v[r, pl.ds(j * 128 + u * 16, 16)]
                        msk = x >= t0
                        plsc.store_compressed(
                            cand_v.at[r, pl.ds(cnt, 16)], x, mask=msk
                        )
                        pc = plsc.all_reduce_population_count(msk)
                        cnt = cnt + pc[0]
                    return cnt

                cnt = lax.fori_loop(0, LKV // 128, c_body, 0)

                neg = jnp.full((16,), -jnp.inf, dtype=jnp.float32)
                cand_v[r, pl.ds(cnt, 16)] = neg
                cand_v[r, pl.ds(cnt + 16, 16)] = neg
                cand_v[r, pl.ds(cnt + 32, 16)] = neg
                cand_v[r, pl.ds(cnt + 48, 16)] = neg

                b0, b1, b2, b3 = ld64(cand_v, 0)
                r0, r1, r2, r3 = _sort64_desc(b0, b1, b2, b3)
                t = jnp.min(r3)
                nb = (cnt + 63) // 64

                def batch_body(bi, carry):
                    r0, r1, r2, r3, t = carry
                    b0, b1, b2, b3 = ld64(cand_v, bi * 64)
                    bmax = jnp.max(
                        jnp.maximum(jnp.maximum(b0, b1), jnp.maximum(b2, b3))
                    )

                    def do_merge(args):
                        r0, r1, r2, r3, b0, b1, b2, b3 = args
                        s0, s1, s2, s3 = _sort64_desc(b0, b1, b2, b3)
                        n0, n1, n2, n3 = _merge_top64(
                            r0, r1, r2, r3, s0, s1, s2, s3
                        )
                        return n0, n1, n2, n3, jnp.min(n3)

                    def no_merge(args):
                        r0, r1, r2, r3, b0, b1, b2, b3 = args
                        return r0, r1, r2, r3, t

                    return lax.cond(
                        bmax > t,
                        do_merge,
                        no_merge,
                        (r0, r1, r2, r3, b0, b1, b2, b3),
                    )

                r0, r1, r2, r3, t = lax.fori_loop(
                    1, nb, batch_body, (r0, r1, r2, r3, t)
                )
                out_v[r, pl.ds(0, 16)] = r0
                out_v[r, pl.ds(16, 16)] = r1
                out_v[r, pl.ds(32, 16)] = r2
                out_v[r, pl.ds(48, 16)] = r3

            pltpu.sync_copy(out_v, out_hbm.at[pl.ds(row0, _SC_CHUNK)])
            return ()

        lax.fori_loop(0, n_chunks, chunk_body, ())

    return topk_kernel(scores, gmax)


def _attend_kernel(s_ref, tk_ref, v_ref, att_ref, sm_ref):
    tv = tk_ref[0]
    m = tv[:, 0:1]
    e = jnp.exp(tv - m)
    denom = jnp.sum(e, axis=1, keepdims=True)
    thr = tv[:, TOPK - 1 :]
    sm_ref[0] = e / denom
    s = s_ref[0]
    p = jnp.where(s >= thr, jnp.exp(s - m), 0.0) / denom
    att_ref[0] = jnp.dot(p, v_ref[0], preferred_element_type=jnp.float32)


def _attend(scores, topk, v3, B, LQ, LKV, block_q=512):
    H, DH = HEADS, HDIM
    BH = B * H
    grid = (BH, LQ // block_q)
    return pl.pallas_call(
        _attend_kernel,
        grid=grid,
        in_specs=[
            pl.BlockSpec((1, block_q, LKV), lambda bh, i: (bh, i, 0)),
            pl.BlockSpec((1, block_q, TOPK), lambda bh, i: (bh, i, 0)),
            pl.BlockSpec(
                (1, LKV, DH), lambda bh, i: (bh % HEADS, bh // HEADS, 0)
            ),
        ],
        out_specs=[
            pl.BlockSpec((1, block_q, DH), lambda bh, i: (bh, i, 0)),
            pl.BlockSpec((1, block_q, TOPK), lambda bh, i: (bh, i, 0)),
        ],
        out_shape=[
            jax.ShapeDtypeStruct((BH, LQ, DH), jnp.float32),
            jax.ShapeDtypeStruct((BH, LQ, TOPK), jnp.float32),
        ],
    )(scores, topk, v3)


def _out_proj_kernel(a_ref, w_ref, b_ref, o_ref):
    h = pl.program_id(2)

    @pl.when(h == 0)
    def _():
        o_ref[...] = jnp.broadcast_to(b_ref[...], o_ref.shape)

    o_ref[...] += jnp.dot(
        a_ref[0, 0], w_ref[0], preferred_element_type=jnp.float32
    )


def _out_proj(att, Wo, bo, B, LQ, block_m=512):
    H, DH, D = HEADS, HDIM, EMBED
    att4 = att.reshape(B, H, LQ, DH)
    wo3 = Wo.reshape(H, DH, D)
    grid = (B, LQ // block_m, H)
    return pl.pallas_call(
        _out_proj_kernel,
        grid=grid,
        in_specs=[
            pl.BlockSpec((1, 1, block_m, DH), lambda b, i, h: (b, h, i, 0)),
            pl.BlockSpec((1, DH, D), lambda b, i, h: (h, 0, 0)),
            pl.BlockSpec((D,), lambda b, i, h: (0,)),
        ],
        out_specs=pl.BlockSpec(
            (1, block_m, D), lambda b, i, h: (b, i, 0)
        ),
        out_shape=jax.ShapeDtypeStruct((B, LQ, D), jnp.float32),
    )(att4, wo3, bo)


def kernel(local_feat, global_feat, Wq, bq, Wk, bk, Wv, bv, Wo, bo):
    B, LQ, D = local_feat.shape
    LKV = global_feat.shape[1]
    H = HEADS

    q3 = _proj_heads(local_feat.reshape(B * LQ, D), Wq, bq)
    k3 = _proj_heads(global_feat.reshape(B * LKV, D), Wk, bk)
    v3 = _proj_heads(global_feat.reshape(B * LKV, D), Wv, bv)

    scores, gmax = _scores(q3, k3, B, LQ, LKV)

    topk = _topk_sc(
        scores.reshape(B * H * LQ, LKV), gmax.reshape(B * H * LQ, _NCELL)
    ).reshape(B * H, LQ, TOPK)

    att, sm = _attend(scores, topk, v3, B, LQ, LKV)

    output = _out_proj(att, Wo, bo, B, LQ)
    return (output, sm.reshape(B, H, LQ, TOPK))


# speedup vs baseline: 2.0698x; 1.0078x over previous
import functools

import jax
import jax.numpy as jnp
from jax import lax
from jax.experimental import pallas as pl
from jax.experimental.pallas import tpu as pltpu
from jax.experimental.pallas import tpu_sc as plsc

EMBED = 1024
HEADS = 16
HDIM = EMBED // HEADS
TOPK = 64


def _matmul_bias_kernel(x_ref, w_ref, b_ref, o_ref):
    o_ref[...] = (
        jnp.dot(x_ref[...], w_ref[...], preferred_element_type=jnp.float32)
        + b_ref[...]
    )


def _matmul_bias(x, w, b, block_m=512):
    m, kdim = x.shape
    n = w.shape[1]
    return pl.pallas_call(
        _matmul_bias_kernel,
        grid=(m // block_m,),
        in_specs=[
            pl.BlockSpec((block_m, kdim), lambda i: (i, 0)),
            pl.BlockSpec((kdim, n), lambda i: (0, 0)),
            pl.BlockSpec((n,), lambda i: (0,)),
        ],
        out_specs=pl.BlockSpec((block_m, n), lambda i: (i, 0)),
        out_shape=jax.ShapeDtypeStruct((m, n), jnp.float32),
    )(x, w, b)


def _proj_heads_kernel(x_ref, w_ref, b_ref, o_ref):
    o_ref[0] = (
        jnp.dot(x_ref[...], w_ref[0], preferred_element_type=jnp.float32)
        + b_ref[0]
    )


def _proj_heads(x, w, b, block_m=512):
    m, D = x.shape
    H, DH = HEADS, HDIM
    w3 = w.reshape(D, H, DH).transpose(1, 0, 2)
    b3 = b.reshape(H, 1, DH)
    return pl.pallas_call(
        _proj_heads_kernel,
        grid=(m // block_m, H),
        in_specs=[
            pl.BlockSpec((block_m, D), lambda i, h: (i, 0)),
            pl.BlockSpec((1, D, DH), lambda i, h: (h, 0, 0)),
            pl.BlockSpec((1, 1, DH), lambda i, h: (h, 0, 0)),
        ],
        out_specs=pl.BlockSpec((1, block_m, DH), lambda i, h: (h, i, 0)),
        out_shape=jax.ShapeDtypeStruct((H, m, DH), jnp.float32),
    )(x, w3, b3)


_NCELL = 128


def _scores_kernel(q_ref, k_ref, o_ref, g_ref, *, scale):
    s = (
        lax.dot_general(
            q_ref[0],
            k_ref[0],
            (((1,), (1,)), ((), ())),
            preferred_element_type=jnp.float32,
        )
        * scale
    )
    o_ref[0] = s
    bq, lkv = s.shape
    gm = lax.slice(s, (0, 0), (bq, _NCELL))
    for k in range(1, lkv // _NCELL):
        gm = jnp.maximum(
            gm, lax.slice(s, (0, k * _NCELL), (bq, (k + 1) * _NCELL))
        )
    g_ref[0] = gm


def _scores(q3, k3, B, LQ, LKV, block_q=256):
    H, DH = HEADS, HDIM
    grid = (B, H, LQ // block_q)
    return pl.pallas_call(
        functools.partial(_scores_kernel, scale=1.0 / (DH**0.5)),
        grid=grid,
        in_specs=[
            pl.BlockSpec(
                (1, block_q, DH),
                lambda b, h, i: (h, b * (LQ // block_q) + i, 0),
            ),
            pl.BlockSpec((1, LKV, DH), lambda b, h, i: (h, b, 0)),
        ],
        out_specs=[
            pl.BlockSpec(
                (1, block_q, LKV), lambda b, h, i: (b * HEADS + h, i, 0)
            ),
            pl.BlockSpec(
                (1, block_q, _NCELL), lambda b, h, i: (b * HEADS + h, i, 0)
            ),
        ],
        out_shape=[
            jax.ShapeDtypeStruct((B * H, LQ, LKV), jnp.float32),
            jax.ShapeDtypeStruct((B * H, LQ, _NCELL), jnp.float32),
        ],
    )(q3, k3)


def _vsort_d(x):
    s, _ = plsc.sort_key_val(x, x, descending=True)
    return s


def _rev(x):
    return lax.rev(x, dimensions=(0,))


def _sort64_desc(b0, b1, b2, b3):
    b0, b1, b2, b3 = _vsort_d(b0), _vsort_d(b1), _vsort_d(b2), _vsort_d(b3)

    def merge16(a, c):
        rc = _rev(c)
        return _vsort_d(jnp.maximum(a, rc)), _vsort_d(jnp.minimum(a, rc))

    x0, x1 = merge16(b0, b1)
    y0, y1 = merge16(b2, b3)
    ry0, ry1 = _rev(y1), _rev(y0)
    hi0, hi1 = jnp.maximum(x0, ry0), jnp.maximum(x1, ry1)
    lo0, lo1 = jnp.minimum(x0, ry0), jnp.minimum(x1, ry1)

    def clean32(p0, p1):
        return (
            _vsort_d(jnp.maximum(p0, p1)),
            _vsort_d(jnp.minimum(p0, p1)),
        )

    r0, r1 = clean32(hi0, hi1)
    r2, r3 = clean32(lo0, lo1)
    return r0, r1, r2, r3


def _merge_top64(r0, r1, r2, r3, s0, s1, s2, s3):
    m0 = jnp.maximum(r0, _rev(s3))
    m1 = jnp.maximum(r1, _rev(s2))
    m2 = jnp.maximum(r2, _rev(s1))
    m3 = jnp.maximum(r3, _rev(s0))
    p0, p1 = jnp.maximum(m0, m2), jnp.maximum(m1, m3)
    q0, q1 = jnp.minimum(m0, m2), jnp.minimum(m1, m3)

    def clean32(a, b):
        return (
            _vsort_d(jnp.maximum(a, b)),
            _vsort_d(jnp.minimum(a, b)),
        )

    r0, r1 = clean32(p0, p1)
    r2, r3 = clean32(q0, q1)
    return r0, r1, r2, r3


_SC_CHUNK = 16
_GROUP = 128


def _topk_sc(scores, gmax):
    R, LKV = scores.shape
    n_gmax = gmax.shape[1]
    mesh = plsc.VectorSubcoreMesh(core_axis_name="c", subcore_axis_name="s")
    info = plsc.get_sparse_core_info()
    n_workers = info.num_cores * info.num_subcores
    rows_per_worker = R // n_workers
    n_chunks = rows_per_worker // _SC_CHUNK

    @functools.partial(
        pl.kernel,
        mesh=mesh,
        out_type=jax.ShapeDtypeStruct((R, TOPK), jnp.float32),
        scratch_types=[
            pltpu.VMEM((_SC_CHUNK, LKV), jnp.float32),
            pltpu.VMEM((_SC_CHUNK, TOPK), jnp.float32),
            pltpu.VMEM((_SC_CHUNK, n_gmax), jnp.float32),
            pltpu.VMEM((_SC_CHUNK, LKV + TOPK), jnp.float32),
        ],
        compiler_params=pltpu.CompilerParams(needs_layout_passes=False),
    )
    def topk_kernel(scores_hbm, gmax_hbm, out_hbm, rows_v, out_v, gmax_v, cand_v):
        wid = lax.axis_index("s") * info.num_cores + lax.axis_index("c")
        base = wid * rows_per_worker

        def chunk_body(c, _):
            row0 = base + c * _SC_CHUNK
            pltpu.sync_copy(scores_hbm.at[pl.ds(row0, _SC_CHUNK)], rows_v)
            pltpu.sync_copy(gmax_hbm.at[pl.ds(row0, _SC_CHUNK)], gmax_v)

            @plsc.parallel_loop(0, _SC_CHUNK, 1, unroll=2)
            def row_body(r):
                def ld64(ref, base_el):
                    return (
                        ref[r, pl.ds(base_el, 16)],
                        ref[r, pl.ds(base_el + 16, 16)],
                        ref[r, pl.ds(base_el + 32, 16)],
                        ref[r, pl.ds(base_el + 48, 16)],
                    )

                g0, g1, g2, g3 = ld64(gmax_v, 0)
                r0, r1, r2, r3 = _sort64_desc(g0, g1, g2, g3)

                def t_body(bi, carry):
                    r0, r1, r2, r3 = carry
                    b0, b1, b2, b3 = ld64(gmax_v, bi * 64)
                    s0, s1, s2, s3 = _sort64_desc(b0, b1, b2, b3)
                    return _merge_top64(r0, r1, r2, r3, s0, s1, s2, s3)

                r0, r1, r2, r3 = lax.fori_loop(
                    1, n_gmax // 64, t_body, (r0, r1, r2, r3)
                )
                t0 = jnp.min(r3)

                def c_body(j, cnt):
                    for u in range(8):
                        x = rows_v[r, pl.ds(j * 128 + u * 16, 16)]
                        msk = x >= t0
                        plsc.store_compressed(
                            cand_v.at[r, pl.ds(cnt, 16)], x, mask=msk
                        )
                        pc = plsc.all_reduce_population_count(msk)
                        cnt = cnt + pc[0]
                    return cnt

                cnt = lax.fori_loop(0, LKV // 128, c_body, 0)

                neg = jnp.full((16,), -jnp.inf, dtype=jnp.float32)
                cand_v[r, pl.ds(cnt, 16)] = neg
                cand_v[r, pl.ds(cnt + 16, 16)] = neg
                cand_v[r, pl.ds(cnt + 32, 16)] = neg
                cand_v[r, pl.ds(cnt + 48, 16)] = neg

                b0, b1, b2, b3 = ld64(cand_v, 0)
                r0, r1, r2, r3 = _sort64_desc(b0, b1, b2, b3)
                t = jnp.min(r3)
                nb = (cnt + 63) // 64

                def batch_body(bi, carry):
                    r0, r1, r2, r3, t = carry
                    b0, b1, b2, b3 = ld64(cand_v, bi * 64)
                    bmax = jnp.max(
                        jnp.maximum(jnp.maximum(b0, b1), jnp.maximum(b2, b3))
                    )

                    def do_merge(args):
                        r0, r1, r2, r3, b0, b1, b2, b3 = args
                        s0, s1, s2, s3 = _sort64_desc(b0, b1, b2, b3)
                        n0, n1, n2, n3 = _merge_top64(
                            r0, r1, r2, r3, s0, s1, s2, s3
                        )
                        return n0, n1, n2, n3, jnp.min(n3)

                    def no_merge(args):
                        r0, r1, r2, r3, b0, b1, b2, b3 = args
                        return r0, r1, r2, r3, t

                    return lax.cond(
                        bmax > t,
                        do_merge,
                        no_merge,
                        (r0, r1, r2, r3, b0, b1, b2, b3),
                    )

                r0, r1, r2, r3, t = lax.fori_loop(
                    1, nb, batch_body, (r0, r1, r2, r3, t)
                )
                out_v[r, pl.ds(0, 16)] = r0
                out_v[r, pl.ds(16, 16)] = r1
                out_v[r, pl.ds(32, 16)] = r2
                out_v[r, pl.ds(48, 16)] = r3

            pltpu.sync_copy(out_v, out_hbm.at[pl.ds(row0, _SC_CHUNK)])
            return ()

        lax.fori_loop(0, n_chunks, chunk_body, ())

    return topk_kernel(scores, gmax)


def _attend_kernel(q_ref, k_ref, tk_ref, v_ref, att_ref, sm_ref, *, scale):
    tv = tk_ref[0]
    m = tv[:, 0:1]
    e = jnp.exp(tv - m)
    denom = jnp.sum(e, axis=1, keepdims=True)
    thr = tv[:, TOPK - 1 :]
    sm_ref[0] = e / denom
    s = (
        lax.dot_general(
            q_ref[0],
            k_ref[0],
            (((1,), (1,)), ((), ())),
            preferred_element_type=jnp.float32,
        )
        * scale
    )
    p = jnp.where(s >= thr, jnp.exp(s - m), 0.0) / denom
    att_ref[0] = jnp.dot(p, v_ref[0], preferred_element_type=jnp.float32)


def _attend(q3, k3, topk, v3, B, LQ, LKV, block_q=512):
    H, DH = HEADS, HDIM
    BH = B * H
    grid = (BH, LQ // block_q)
    nblk = LQ // block_q
    return pl.pallas_call(
        functools.partial(_attend_kernel, scale=1.0 / (DH**0.5)),
        grid=grid,
        in_specs=[
            pl.BlockSpec(
                (1, block_q, DH),
                lambda bh, i: (bh % HEADS, (bh // HEADS) * nblk + i, 0),
            ),
            pl.BlockSpec(
                (1, LKV, DH), lambda bh, i: (bh % HEADS, bh // HEADS, 0)
            ),
            pl.BlockSpec((1, block_q, TOPK), lambda bh, i: (bh, i, 0)),
            pl.BlockSpec(
                (1, LKV, DH), lambda bh, i: (bh % HEADS, bh // HEADS, 0)
            ),
        ],
        out_specs=[
            pl.BlockSpec((1, block_q, DH), lambda bh, i: (bh, i, 0)),
            pl.BlockSpec((1, block_q, TOPK), lambda bh, i: (bh, i, 0)),
        ],
        out_shape=[
            jax.ShapeDtypeStruct((BH, LQ, DH), jnp.float32),
            jax.ShapeDtypeStruct((BH, LQ, TOPK), jnp.float32),
        ],
    )(q3, k3, topk, v3)


def _out_proj_kernel(a_ref, w_ref, b_ref, o_ref):
    h = pl.program_id(2)

    @pl.when(h == 0)
    def _():
        o_ref[...] = jnp.broadcast_to(b_ref[...], o_ref.shape)

    o_ref[...] += jnp.dot(
        a_ref[0, 0], w_ref[0], preferred_element_type=jnp.float32
    )


def _out_proj(att, Wo, bo, B, LQ, block_m=512):
    H, DH, D = HEADS, HDIM, EMBED
    att4 = att.reshape(B, H, LQ, DH)
    wo3 = Wo.reshape(H, DH, D)
    grid = (B, LQ // block_m, H)
    return pl.pallas_call(
        _out_proj_kernel,
        grid=grid,
        in_specs=[
            pl.BlockSpec((1, 1, block_m, DH), lambda b, i, h: (b, h, i, 0)),
            pl.BlockSpec((1, DH, D), lambda b, i, h: (h, 0, 0)),
            pl.BlockSpec((D,), lambda b, i, h: (0,)),
        ],
        out_specs=pl.BlockSpec(
            (1, block_m, D), lambda b, i, h: (b, i, 0)
        ),
        out_shape=jax.ShapeDtypeStruct((B, LQ, D), jnp.float32),
    )(att4, wo3, bo)


def kernel(local_feat, global_feat, Wq, bq, Wk, bk, Wv, bv, Wo, bo):
    B, LQ, D = local_feat.shape
    LKV = global_feat.shape[1]
    H = HEADS

    q3 = _proj_heads(local_feat.reshape(B * LQ, D), Wq, bq)
    k3 = _proj_heads(global_feat.reshape(B * LKV, D), Wk, bk)
    v3 = _proj_heads(global_feat.reshape(B * LKV, D), Wv, bv)

    scores, gmax = _scores(q3, k3, B, LQ, LKV)

    topk = _topk_sc(
        scores.reshape(B * H * LQ, LKV), gmax.reshape(B * H * LQ, _NCELL)
    ).reshape(B * H, LQ, TOPK)

    att, sm = _attend(q3, k3, topk, v3, B, LQ, LKV)

    output = _out_proj(att, Wo, bo, B, LQ)
    return (output, sm.reshape(B, H, LQ, TOPK))


# speedup vs baseline: 3.3005x; 1.5946x over previous
import functools

import jax
import jax.numpy as jnp
from jax import lax
from jax.experimental import pallas as pl
from jax.experimental.pallas import tpu as pltpu
from jax.experimental.pallas import tpu_sc as plsc

EMBED = 1024
HEADS = 16
HDIM = EMBED // HEADS
TOPK = 64


def _matmul_bias_kernel(x_ref, w_ref, b_ref, o_ref):
    o_ref[...] = (
        jnp.dot(x_ref[...], w_ref[...], preferred_element_type=jnp.float32)
        + b_ref[...]
    )


def _matmul_bias(x, w, b, block_m=512):
    m, kdim = x.shape
    n = w.shape[1]
    return pl.pallas_call(
        _matmul_bias_kernel,
        grid=(m // block_m,),
        in_specs=[
            pl.BlockSpec((block_m, kdim), lambda i: (i, 0)),
            pl.BlockSpec((kdim, n), lambda i: (0, 0)),
            pl.BlockSpec((n,), lambda i: (0,)),
        ],
        out_specs=pl.BlockSpec((block_m, n), lambda i: (i, 0)),
        out_shape=jax.ShapeDtypeStruct((m, n), jnp.float32),
    )(x, w, b)


def _proj_heads_kernel(x_ref, w_ref, b_ref, o_ref):
    o_ref[0] = (
        jnp.dot(x_ref[...], w_ref[0], preferred_element_type=jnp.float32)
        + b_ref[0]
    )


def _proj_heads(x, w, b, block_m=512):
    m, D = x.shape
    H, DH = HEADS, HDIM
    w3 = w.reshape(D, H, DH).transpose(1, 0, 2)
    b3 = b.reshape(H, 1, DH)
    return pl.pallas_call(
        _proj_heads_kernel,
        grid=(m // block_m, H),
        in_specs=[
            pl.BlockSpec((block_m, D), lambda i, h: (i, 0)),
            pl.BlockSpec((1, D, DH), lambda i, h: (h, 0, 0)),
            pl.BlockSpec((1, 1, DH), lambda i, h: (h, 0, 0)),
        ],
        out_specs=pl.BlockSpec((1, block_m, DH), lambda i, h: (h, i, 0)),
        out_shape=jax.ShapeDtypeStruct((H, m, DH), jnp.float32),
    )(x, w3, b3)


_NCELL = 128


def _scores_kernel(q_ref, k_ref, o_ref, g_ref, *, scale):
    s = (
        lax.dot_general(
            q_ref[0],
            k_ref[0],
            (((1,), (1,)), ((), ())),
            preferred_element_type=jnp.float32,
        )
        * scale
    )
    o_ref[0] = s
    bq, lkv = s.shape
    gm = lax.slice(s, (0, 0), (bq, _NCELL))
    for k in range(1, lkv // _NCELL):
        gm = jnp.maximum(
            gm, lax.slice(s, (0, k * _NCELL), (bq, (k + 1) * _NCELL))
        )
    g_ref[0] = gm


def _scores(q3, k3, B, LQ, LKV, block_q=256):
    H, DH = HEADS, HDIM
    grid = (B, H, LQ // block_q)
    return pl.pallas_call(
        functools.partial(_scores_kernel, scale=1.0 / (DH**0.5)),
        grid=grid,
        in_specs=[
            pl.BlockSpec(
                (1, block_q, DH),
                lambda b, h, i: (h, b * (LQ // block_q) + i, 0),
            ),
            pl.BlockSpec((1, LKV, DH), lambda b, h, i: (h, b, 0)),
        ],
        out_specs=[
            pl.BlockSpec(
                (1, block_q, LKV), lambda b, h, i: (b * HEADS + h, i, 0)
            ),
            pl.BlockSpec(
                (1, block_q, _NCELL), lambda b, h, i: (b * HEADS + h, i, 0)
            ),
        ],
        out_shape=[
            jax.ShapeDtypeStruct((B * H, LQ, LKV), jnp.float32),
            jax.ShapeDtypeStruct((B * H, LQ, _NCELL), jnp.float32),
        ],
    )(q3, k3)


def _vsort_d(x):
    s, _ = plsc.sort_key_val(x, x, descending=True)
    return s


def _rev(x):
    return lax.rev(x, dimensions=(0,))


def _sort64_desc(b0, b1, b2, b3):
    b0, b1, b2, b3 = _vsort_d(b0), _vsort_d(b1), _vsort_d(b2), _vsort_d(b3)

    def merge16(a, c):
        rc = _rev(c)
        return _vsort_d(jnp.maximum(a, rc)), _vsort_d(jnp.minimum(a, rc))

    x0, x1 = merge16(b0, b1)
    y0, y1 = merge16(b2, b3)
    ry0, ry1 = _rev(y1), _rev(y0)
    hi0, hi1 = jnp.maximum(x0, ry0), jnp.maximum(x1, ry1)
    lo0, lo1 = jnp.minimum(x0, ry0), jnp.minimum(x1, ry1)

    def clean32(p0, p1):
        return (
            _vsort_d(jnp.maximum(p0, p1)),
            _vsort_d(jnp.minimum(p0, p1)),
        )

    r0, r1 = clean32(hi0, hi1)
    r2, r3 = clean32(lo0, lo1)
    return r0, r1, r2, r3


def _merge_top64(r0, r1, r2, r3, s0, s1, s2, s3):
    m0 = jnp.maximum(r0, _rev(s3))
    m1 = jnp.maximum(r1, _rev(s2))
    m2 = jnp.maximum(r2, _rev(s1))
    m3 = jnp.maximum(r3, _rev(s0))
    p0, p1 = jnp.maximum(m0, m2), jnp.maximum(m1, m3)
    q0, q1 = jnp.minimum(m0, m2), jnp.minimum(m1, m3)

    def clean32(a, b):
        return (
            _vsort_d(jnp.maximum(a, b)),
            _vsort_d(jnp.minimum(a, b)),
        )

    r0, r1 = clean32(p0, p1)
    r2, r3 = clean32(q0, q1)
    return r0, r1, r2, r3


_SC_CHUNK = 16
_GROUP = 128


def _topk_sc(scores, gmax):
    R, LKV = scores.shape
    n_gmax = gmax.shape[1]
    mesh = plsc.VectorSubcoreMesh(core_axis_name="c", subcore_axis_name="s")
    info = plsc.get_sparse_core_info()
    n_workers = info.num_cores * info.num_subcores
    rows_per_worker = R // n_workers
    n_chunks = rows_per_worker // _SC_CHUNK

    @functools.partial(
        pl.kernel,
        mesh=mesh,
        out_type=jax.ShapeDtypeStruct((R, TOPK), jnp.float32),
        scratch_types=[
            pltpu.VMEM((_SC_CHUNK, LKV), jnp.float32),
            pltpu.VMEM((_SC_CHUNK, TOPK), jnp.float32),
            pltpu.VMEM((_SC_CHUNK, n_gmax), jnp.float32),
            pltpu.VMEM((_SC_CHUNK, LKV + TOPK), jnp.float32),
        ],
        compiler_params=pltpu.CompilerParams(needs_layout_passes=False),
    )
    def topk_kernel(scores_hbm, gmax_hbm, out_hbm, rows_v, out_v, gmax_v, cand_v):
        wid = lax.axis_index("s") * info.num_cores + lax.axis_index("c")
        base = wid * rows_per_worker

        def chunk_body(c, _):
            row0 = base + c * _SC_CHUNK
            pltpu.sync_copy(scores_hbm.at[pl.ds(row0, _SC_CHUNK)], rows_v)
            pltpu.sync_copy(gmax_hbm.at[pl.ds(row0, _SC_CHUNK)], gmax_v)

            @plsc.parallel_loop(0, _SC_CHUNK, 1, unroll=2)
            def row_body(r):
                def ld64(ref, base_el):
                    return (
                        ref[r, pl.ds(base_el, 16)],
                        ref[r, pl.ds(base_el + 16, 16)],
                        ref[r, pl.ds(base_el + 32, 16)],
                        ref[r, pl.ds(base_el + 48, 16)],
                    )

                g0, g1, g2, g3 = ld64(gmax_v, 0)
                r0, r1, r2, r3 = _sort64_desc(g0, g1, g2, g3)

                def t_body(bi, carry):
                    r0, r1, r2, r3 = carry
                    b0, b1, b2, b3 = ld64(gmax_v, bi * 64)
                    s0, s1, s2, s3 = _sort64_desc(b0, b1, b2, b3)
                    return _merge_top64(r0, r1, r2, r3, s0, s1, s2, s3)

                r0, r1, r2, r3 = lax.fori_loop(
                    1, n_gmax // 64, t_body, (r0, r1, r2, r3)
                )
                t0 = jnp.min(r3)

                def c_body(j, cnt):
                    xs, msks, pcs = [], [], []
                    for u in range(8):
                        x = rows_v[r, pl.ds(j * 128 + u * 16, 16)]
                        msk = x >= t0
                        xs.append(x)
                        msks.append(msk)
                        pcs.append(
                            plsc.all_reduce_population_count(msk)[0]
                        )
                    offs = [cnt]
                    for u in range(8):
                        offs.append(offs[-1] + pcs[u])
                    for u in range(8):
                        plsc.store_compressed(
                            cand_v.at[r, pl.ds(offs[u], 16)],
                            xs[u],
                            mask=msks[u],
                        )
                    return offs[-1]

                cnt = lax.fori_loop(0, LKV // 128, c_body, 0)

                neg = jnp.full((16,), -jnp.inf, dtype=jnp.float32)
                cand_v[r, pl.ds(cnt, 16)] = neg
                cand_v[r, pl.ds(cnt + 16, 16)] = neg
                cand_v[r, pl.ds(cnt + 32, 16)] = neg
                cand_v[r, pl.ds(cnt + 48, 16)] = neg

                b0, b1, b2, b3 = ld64(cand_v, 0)
                r0, r1, r2, r3 = _sort64_desc(b0, b1, b2, b3)
                t = jnp.min(r3)
                nb = (cnt + 63) // 64

                def batch_body(bi, carry):
                    r0, r1, r2, r3, t = carry
                    b0, b1, b2, b3 = ld64(cand_v, bi * 64)
                    bmax = jnp.max(
                        jnp.maximum(jnp.maximum(b0, b1), jnp.maximum(b2, b3))
                    )

                    def do_merge(args):
                        r0, r1, r2, r3, b0, b1, b2, b3 = args
                        s0, s1, s2, s3 = _sort64_desc(b0, b1, b2, b3)
                        n0, n1, n2, n3 = _merge_top64(
                            r0, r1, r2, r3, s0, s1, s2, s3
                        )
                        return n0, n1, n2, n3, jnp.min(n3)

                    def no_merge(args):
                        r0, r1, r2, r3, b0, b1, b2, b3 = args
                        return r0, r1, r2, r3, t

                    return lax.cond(
                        bmax > t,
                        do_merge,
                        no_merge,
                        (r0, r1, r2, r3, b0, b1, b2, b3),
                    )

                r0, r1, r2, r3, t = lax.fori_loop(
                    1, nb, batch_body, (r0, r1, r2, r3, t)
                )
                out_v[r, pl.ds(0, 16)] = r0
                out_v[r, pl.ds(16, 16)] = r1
                out_v[r, pl.ds(32, 16)] = r2
                out_v[r, pl.ds(48, 16)] = r3

            pltpu.sync_copy(out_v, out_hbm.at[pl.ds(row0, _SC_CHUNK)])
            return ()

        lax.fori_loop(0, n_chunks, chunk_body, ())

    return topk_kernel(scores, gmax)


def _attend_kernel(q_ref, k_ref, tk_ref, v_ref, att_ref, sm_ref, *, scale):
    tv = tk_ref[0]
    m = tv[:, 0:1]
    e = jnp.exp(tv - m)
    denom = jnp.sum(e, axis=1, keepdims=True)
    thr = tv[:, TOPK - 1 :]
    sm_ref[0] = e / denom
    s = (
        lax.dot_general(
            q_ref[0],
            k_ref[0],
            (((1,), (1,)), ((), ())),
            preferred_element_type=jnp.float32,
        )
        * scale
    )
    p = jnp.where(s >= thr, jnp.exp(s - m), 0.0) / denom
    att_ref[0] = jnp.dot(p, v_ref[0], preferred_element_type=jnp.float32)


def _attend(q3, k3, topk, v3, B, LQ, LKV, block_q=512):
    H, DH = HEADS, HDIM
    BH = B * H
    grid = (BH, LQ // block_q)
    nblk = LQ // block_q
    return pl.pallas_call(
        functools.partial(_attend_kernel, scale=1.0 / (DH**0.5)),
        grid=grid,
        in_specs=[
            pl.BlockSpec(
                (1, block_q, DH),
                lambda bh, i: (bh % HEADS, (bh // HEADS) * nblk + i, 0),
            ),
            pl.BlockSpec(
                (1, LKV, DH), lambda bh, i: (bh % HEADS, bh // HEADS, 0)
            ),
            pl.BlockSpec((1, block_q, TOPK), lambda bh, i: (bh, i, 0)),
            pl.BlockSpec(
                (1, LKV, DH), lambda bh, i: (bh % HEADS, bh // HEADS, 0)
            ),
        ],
        out_specs=[
            pl.BlockSpec((1, block_q, DH), lambda bh, i: (bh, i, 0)),
            pl.BlockSpec((1, block_q, TOPK), lambda bh, i: (bh, i, 0)),
        ],
        out_shape=[
            jax.ShapeDtypeStruct((BH, LQ, DH), jnp.float32),
            jax.ShapeDtypeStruct((BH, LQ, TOPK), jnp.float32),
        ],
    )(q3, k3, topk, v3)


def _out_proj_kernel(a_ref, w_ref, b_ref, o_ref):
    h = pl.program_id(2)

    @pl.when(h == 0)
    def _():
        o_ref[...] = jnp.broadcast_to(b_ref[...], o_ref.shape)

    o_ref[...] += jnp.dot(
        a_ref[0, 0], w_ref[0], preferred_element_type=jnp.float32
    )


def _out_proj(att, Wo, bo, B, LQ, block_m=512):
    H, DH, D = HEADS, HDIM, EMBED
    att4 = att.reshape(B, H, LQ, DH)
    wo3 = Wo.reshape(H, DH, D)
    grid = (B, LQ // block_m, H)
    return pl.pallas_call(
        _out_proj_kernel,
        grid=grid,
        in_specs=[
            pl.BlockSpec((1, 1, block_m, DH), lambda b, i, h: (b, h, i, 0)),
            pl.BlockSpec((1, DH, D), lambda b, i, h: (h, 0, 0)),
            pl.BlockSpec((D,), lambda b, i, h: (0,)),
        ],
        out_specs=pl.BlockSpec(
            (1, block_m, D), lambda b, i, h: (b, i, 0)
        ),
        out_shape=jax.ShapeDtypeStruct((B, LQ, D), jnp.float32),
    )(att4, wo3, bo)


def kernel(local_feat, global_feat, Wq, bq, Wk, bk, Wv, bv, Wo, bo):
    B, LQ, D = local_feat.shape
    LKV = global_feat.shape[1]
    H = HEADS

    q3 = _proj_heads(local_feat.reshape(B * LQ, D), Wq, bq)
    k3 = _proj_heads(global_feat.reshape(B * LKV, D), Wk, bk)
    v3 = _proj_heads(global_feat.reshape(B * LKV, D), Wv, bv)

    scores, gmax = _scores(q3, k3, B, LQ, LKV)

    topk = _topk_sc(
        scores.reshape(B * H * LQ, LKV), gmax.reshape(B * H * LQ, _NCELL)
    ).reshape(B * H, LQ, TOPK)

    att, sm = _attend(q3, k3, topk, v3, B, LQ, LKV)

    output = _out_proj(att, Wo, bo, B, LQ)
    return (output, sm.reshape(B, H, LQ, TOPK))


# speedup vs baseline: 3.3052x; 1.0014x over previous
import functools

import jax
import jax.numpy as jnp
from jax import lax
from jax.experimental import pallas as pl
from jax.experimental.pallas import tpu as pltpu
from jax.experimental.pallas import tpu_sc as plsc

EMBED = 1024
HEADS = 16
HDIM = EMBED // HEADS
TOPK = 64


def _matmul_bias_kernel(x_ref, w_ref, b_ref, o_ref):
    o_ref[...] = (
        jnp.dot(x_ref[...], w_ref[...], preferred_element_type=jnp.float32)
        + b_ref[...]
    )


def _matmul_bias(x, w, b, block_m=512):
    m, kdim = x.shape
    n = w.shape[1]
    return pl.pallas_call(
        _matmul_bias_kernel,
        grid=(m // block_m,),
        in_specs=[
            pl.BlockSpec((block_m, kdim), lambda i: (i, 0)),
            pl.BlockSpec((kdim, n), lambda i: (0, 0)),
            pl.BlockSpec((n,), lambda i: (0,)),
        ],
        out_specs=pl.BlockSpec((block_m, n), lambda i: (i, 0)),
        out_shape=jax.ShapeDtypeStruct((m, n), jnp.float32),
    )(x, w, b)


def _proj_heads_kernel(x_ref, w_ref, b_ref, o_ref):
    o_ref[0] = (
        jnp.dot(x_ref[...], w_ref[0], preferred_element_type=jnp.float32)
        + b_ref[0]
    )


def _proj_heads(x, w, b, block_m=512):
    m, D = x.shape
    H, DH = HEADS, HDIM
    w3 = w.reshape(D, H, DH).transpose(1, 0, 2)
    b3 = b.reshape(H, 1, DH)
    return pl.pallas_call(
        _proj_heads_kernel,
        grid=(m // block_m, H),
        in_specs=[
            pl.BlockSpec((block_m, D), lambda i, h: (i, 0)),
            pl.BlockSpec((1, D, DH), lambda i, h: (h, 0, 0)),
            pl.BlockSpec((1, 1, DH), lambda i, h: (h, 0, 0)),
        ],
        out_specs=pl.BlockSpec((1, block_m, DH), lambda i, h: (h, i, 0)),
        out_shape=jax.ShapeDtypeStruct((H, m, DH), jnp.float32),
    )(x, w3, b3)


_NCELL = 128


def _scores_kernel(q_ref, k_ref, o_ref, g_ref, *, scale):
    s = (
        lax.dot_general(
            q_ref[0],
            k_ref[0],
            (((1,), (1,)), ((), ())),
            preferred_element_type=jnp.float32,
        )
        * scale
    )
    o_ref[0] = s
    bq, lkv = s.shape
    gm = lax.slice(s, (0, 0), (bq, _NCELL))
    for k in range(1, lkv // _NCELL):
        gm = jnp.maximum(
            gm, lax.slice(s, (0, k * _NCELL), (bq, (k + 1) * _NCELL))
        )
    g_ref[0] = gm


def _scores(q3, k3, B, LQ, LKV, block_q=256):
    H, DH = HEADS, HDIM
    grid = (B, H, LQ // block_q)
    return pl.pallas_call(
        functools.partial(_scores_kernel, scale=1.0 / (DH**0.5)),
        grid=grid,
        in_specs=[
            pl.BlockSpec(
                (1, block_q, DH),
                lambda b, h, i: (h, b * (LQ // block_q) + i, 0),
            ),
            pl.BlockSpec((1, LKV, DH), lambda b, h, i: (h, b, 0)),
        ],
        out_specs=[
            pl.BlockSpec(
                (1, block_q, LKV), lambda b, h, i: (b * HEADS + h, i, 0)
            ),
            pl.BlockSpec(
                (1, block_q, _NCELL), lambda b, h, i: (b * HEADS + h, i, 0)
            ),
        ],
        out_shape=[
            jax.ShapeDtypeStruct((B * H, LQ, LKV), jnp.float32),
            jax.ShapeDtypeStruct((B * H, LQ, _NCELL), jnp.float32),
        ],
    )(q3, k3)


def _vsort_d(x):
    s, _ = plsc.sort_key_val(x, x, descending=True)
    return s


def _rev(x):
    return lax.rev(x, dimensions=(0,))


def _sort64_desc(b0, b1, b2, b3):
    b0, b1, b2, b3 = _vsort_d(b0), _vsort_d(b1), _vsort_d(b2), _vsort_d(b3)

    def merge16(a, c):
        rc = _rev(c)
        return _vsort_d(jnp.maximum(a, rc)), _vsort_d(jnp.minimum(a, rc))

    x0, x1 = merge16(b0, b1)
    y0, y1 = merge16(b2, b3)
    ry0, ry1 = _rev(y1), _rev(y0)
    hi0, hi1 = jnp.maximum(x0, ry0), jnp.maximum(x1, ry1)
    lo0, lo1 = jnp.minimum(x0, ry0), jnp.minimum(x1, ry1)

    def clean32(p0, p1):
        return (
            _vsort_d(jnp.maximum(p0, p1)),
            _vsort_d(jnp.minimum(p0, p1)),
        )

    r0, r1 = clean32(hi0, hi1)
    r2, r3 = clean32(lo0, lo1)
    return r0, r1, r2, r3


def _merge_top64(r0, r1, r2, r3, s0, s1, s2, s3):
    m0 = jnp.maximum(r0, _rev(s3))
    m1 = jnp.maximum(r1, _rev(s2))
    m2 = jnp.maximum(r2, _rev(s1))
    m3 = jnp.maximum(r3, _rev(s0))
    p0, p1 = jnp.maximum(m0, m2), jnp.maximum(m1, m3)
    q0, q1 = jnp.minimum(m0, m2), jnp.minimum(m1, m3)

    def clean32(a, b):
        return (
            _vsort_d(jnp.maximum(a, b)),
            _vsort_d(jnp.minimum(a, b)),
        )

    r0, r1 = clean32(p0, p1)
    r2, r3 = clean32(q0, q1)
    return r0, r1, r2, r3


_SC_CHUNK = 16
_GROUP = 128


def _topk_sc(scores, gmax):
    R, LKV = scores.shape
    n_gmax = gmax.shape[1]
    mesh = plsc.VectorSubcoreMesh(core_axis_name="c", subcore_axis_name="s")
    info = plsc.get_sparse_core_info()
    n_workers = info.num_cores * info.num_subcores
    rows_per_worker = R // n_workers
    n_chunks = rows_per_worker // _SC_CHUNK

    @functools.partial(
        pl.kernel,
        mesh=mesh,
        out_type=jax.ShapeDtypeStruct((R, TOPK), jnp.float32),
        scratch_types=[
            pltpu.VMEM((_SC_CHUNK, LKV), jnp.float32),
            pltpu.VMEM((_SC_CHUNK, TOPK), jnp.float32),
            pltpu.VMEM((_SC_CHUNK, n_gmax), jnp.float32),
            pltpu.VMEM((_SC_CHUNK, LKV + TOPK), jnp.float32),
        ],
        compiler_params=pltpu.CompilerParams(needs_layout_passes=False),
    )
    def topk_kernel(scores_hbm, gmax_hbm, out_hbm, rows_v, out_v, gmax_v, cand_v):
        wid = lax.axis_index("s") * info.num_cores + lax.axis_index("c")
        base = wid * rows_per_worker

        def chunk_body(c, _):
            row0 = base + c * _SC_CHUNK
            pltpu.sync_copy(scores_hbm.at[pl.ds(row0, _SC_CHUNK)], rows_v)
            pltpu.sync_copy(gmax_hbm.at[pl.ds(row0, _SC_CHUNK)], gmax_v)

            @plsc.parallel_loop(0, _SC_CHUNK, 1, unroll=3)
            def row_body(r):
                def ld64(ref, base_el):
                    return (
                        ref[r, pl.ds(base_el, 16)],
                        ref[r, pl.ds(base_el + 16, 16)],
                        ref[r, pl.ds(base_el + 32, 16)],
                        ref[r, pl.ds(base_el + 48, 16)],
                    )

                g0, g1, g2, g3 = ld64(gmax_v, 0)
                r0, r1, r2, r3 = _sort64_desc(g0, g1, g2, g3)

                def t_body(bi, carry):
                    r0, r1, r2, r3 = carry
                    b0, b1, b2, b3 = ld64(gmax_v, bi * 64)
                    s0, s1, s2, s3 = _sort64_desc(b0, b1, b2, b3)
                    return _merge_top64(r0, r1, r2, r3, s0, s1, s2, s3)

                r0, r1, r2, r3 = lax.fori_loop(
                    1, n_gmax // 64, t_body, (r0, r1, r2, r3)
                )
                t0 = jnp.min(r3)

                def c_body(j, cnt):
                    xs, msks, pcs = [], [], []
                    for u in range(8):
                        x = rows_v[r, pl.ds(j * 128 + u * 16, 16)]
                        msk = x >= t0
                        xs.append(x)
                        msks.append(msk)
                        pcs.append(
                            plsc.all_reduce_population_count(msk)[0]
                        )
                    offs = [cnt]
                    for u in range(8):
                        offs.append(offs[-1] + pcs[u])
                    for u in range(8):
                        plsc.store_compressed(
                            cand_v.at[r, pl.ds(offs[u], 16)],
                            xs[u],
                            mask=msks[u],
                        )
                    return offs[-1]

                cnt = lax.fori_loop(0, LKV // 128, c_body, 0)

                neg = jnp.full((16,), -jnp.inf, dtype=jnp.float32)
                cand_v[r, pl.ds(cnt, 16)] = neg
                cand_v[r, pl.ds(cnt + 16, 16)] = neg
                cand_v[r, pl.ds(cnt + 32, 16)] = neg
                cand_v[r, pl.ds(cnt + 48, 16)] = neg

                b0, b1, b2, b3 = ld64(cand_v, 0)
                r0, r1, r2, r3 = _sort64_desc(b0, b1, b2, b3)
                t = jnp.min(r3)
                nb = (cnt + 63) // 64

                def batch_body(bi, carry):
                    r0, r1, r2, r3, t = carry
                    b0, b1, b2, b3 = ld64(cand_v, bi * 64)
                    bmax = jnp.max(
                        jnp.maximum(jnp.maximum(b0, b1), jnp.maximum(b2, b3))
                    )

                    def do_merge(args):
                        r0, r1, r2, r3, b0, b1, b2, b3 = args
                        s0, s1, s2, s3 = _sort64_desc(b0, b1, b2, b3)
                        n0, n1, n2, n3 = _merge_top64(
                            r0, r1, r2, r3, s0, s1, s2, s3
                        )
                        return n0, n1, n2, n3, jnp.min(n3)

                    def no_merge(args):
                        r0, r1, r2, r3, b0, b1, b2, b3 = args
                        return r0, r1, r2, r3, t

                    return lax.cond(
                        bmax > t,
                        do_merge,
                        no_merge,
                        (r0, r1, r2, r3, b0, b1, b2, b3),
                    )

                r0, r1, r2, r3, t = lax.fori_loop(
                    1, nb, batch_body, (r0, r1, r2, r3, t)
                )
                out_v[r, pl.ds(0, 16)] = r0
                out_v[r, pl.ds(16, 16)] = r1
                out_v[r, pl.ds(32, 16)] = r2
                out_v[r, pl.ds(48, 16)] = r3

            pltpu.sync_copy(out_v, out_hbm.at[pl.ds(row0, _SC_CHUNK)])
            return ()

        lax.fori_loop(0, n_chunks, chunk_body, ())

    return topk_kernel(scores, gmax)


def _attend_kernel(q_ref, k_ref, tk_ref, v_ref, att_ref, sm_ref, *, scale):
    tv = tk_ref[0]
    m = tv[:, 0:1]
    e = jnp.exp(tv - m)
    denom = jnp.sum(e, axis=1, keepdims=True)
    thr = tv[:, TOPK - 1 :]
    sm_ref[0] = e / denom
    s = (
        lax.dot_general(
            q_ref[0],
            k_ref[0],
            (((1,), (1,)), ((), ())),
            preferred_element_type=jnp.float32,
        )
        * scale
    )
    p = jnp.where(s >= thr, jnp.exp(s - m), 0.0) / denom
    att_ref[0] = jnp.dot(p, v_ref[0], preferred_element_type=jnp.float32)


def _attend(q3, k3, topk, v3, B, LQ, LKV, block_q=512):
    H, DH = HEADS, HDIM
    BH = B * H
    grid = (BH, LQ // block_q)
    nblk = LQ // block_q
    return pl.pallas_call(
        functools.partial(_attend_kernel, scale=1.0 / (DH**0.5)),
        grid=grid,
        in_specs=[
            pl.BlockSpec(
                (1, block_q, DH),
                lambda bh, i: (bh % HEADS, (bh // HEADS) * nblk + i, 0),
            ),
            pl.BlockSpec(
                (1, LKV, DH), lambda bh, i: (bh % HEADS, bh // HEADS, 0)
            ),
            pl.BlockSpec((1, block_q, TOPK), lambda bh, i: (bh, i, 0)),
            pl.BlockSpec(
                (1, LKV, DH), lambda bh, i: (bh % HEADS, bh // HEADS, 0)
            ),
        ],
        out_specs=[
            pl.BlockSpec((1, block_q, DH), lambda bh, i: (bh, i, 0)),
            pl.BlockSpec((1, block_q, TOPK), lambda bh, i: (bh, i, 0)),
        ],
        out_shape=[
            jax.ShapeDtypeStruct((BH, LQ, DH), jnp.float32),
            jax.ShapeDtypeStruct((BH, LQ, TOPK), jnp.float32),
        ],
    )(q3, k3, topk, v3)


def _out_proj_kernel(a_ref, w_ref, b_ref, o_ref):
    h = pl.program_id(2)

    @pl.when(h == 0)
    def _():
        o_ref[...] = jnp.broadcast_to(b_ref[...], o_ref.shape)

    o_ref[...] += jnp.dot(
        a_ref[0, 0], w_ref[0], preferred_element_type=jnp.float32
    )


def _out_proj(att, Wo, bo, B, LQ, block_m=512):
    H, DH, D = HEADS, HDIM, EMBED
    att4 = att.reshape(B, H, LQ, DH)
    wo3 = Wo.reshape(H, DH, D)
    grid = (B, LQ // block_m, H)
    return pl.pallas_call(
        _out_proj_kernel,
        grid=grid,
        in_specs=[
            pl.BlockSpec((1, 1, block_m, DH), lambda b, i, h: (b, h, i, 0)),
            pl.BlockSpec((1, DH, D), lambda b, i, h: (h, 0, 0)),
            pl.BlockSpec((D,), lambda b, i, h: (0,)),
        ],
        out_specs=pl.BlockSpec(
            (1, block_m, D), lambda b, i, h: (b, i, 0)
        ),
        out_shape=jax.ShapeDtypeStruct((B, LQ, D), jnp.float32),
    )(att4, wo3, bo)


def kernel(local_feat, global_feat, Wq, bq, Wk, bk, Wv, bv, Wo, bo):
    B, LQ, D = local_feat.shape
    LKV = global_feat.shape[1]
    H = HEADS

    q3 = _proj_heads(local_feat.reshape(B * LQ, D), Wq, bq)
    k3 = _proj_heads(global_feat.reshape(B * LKV, D), Wk, bk)
    v3 = _proj_heads(global_feat.reshape(B * LKV, D), Wv, bv)

    scores, gmax = _scores(q3, k3, B, LQ, LKV)

    topk = _topk_sc(
        scores.reshape(B * H * LQ, LKV), gmax.reshape(B * H * LQ, _NCELL)
    ).reshape(B * H, LQ, TOPK)

    att, sm = _attend(q3, k3, topk, v3, B, LQ, LKV)

    output = _out_proj(att, Wo, bo, B, LQ)
    return (output, sm.reshape(B, H, LQ, TOPK))


# speedup vs baseline: 3.4063x; 1.0306x over previous
import functools

import jax
import jax.numpy as jnp
from jax import lax
from jax.experimental import pallas as pl
from jax.experimental.pallas import tpu as pltpu
from jax.experimental.pallas import tpu_sc as plsc

EMBED = 1024
HEADS = 16
HDIM = EMBED // HEADS
TOPK = 64


def _matmul_bias_kernel(x_ref, w_ref, b_ref, o_ref):
    o_ref[...] = (
        jnp.dot(x_ref[...], w_ref[...], preferred_element_type=jnp.float32)
        + b_ref[...]
    )


def _matmul_bias(x, w, b, block_m=512):
    m, kdim = x.shape
    n = w.shape[1]
    return pl.pallas_call(
        _matmul_bias_kernel,
        grid=(m // block_m,),
        in_specs=[
            pl.BlockSpec((block_m, kdim), lambda i: (i, 0)),
            pl.BlockSpec((kdim, n), lambda i: (0, 0)),
            pl.BlockSpec((n,), lambda i: (0,)),
        ],
        out_specs=pl.BlockSpec((block_m, n), lambda i: (i, 0)),
        out_shape=jax.ShapeDtypeStruct((m, n), jnp.float32),
    )(x, w, b)


def _proj_heads_kernel(x_ref, w_ref, b_ref, o_ref):
    o_ref[0] = (
        jnp.dot(x_ref[...], w_ref[0], preferred_element_type=jnp.float32)
        + b_ref[0]
    )


def _proj_heads(x, w, b, block_m=512):
    m, D = x.shape
    H, DH = HEADS, HDIM
    w3 = w.reshape(D, H, DH).transpose(1, 0, 2)
    b3 = b.reshape(H, 1, DH)
    return pl.pallas_call(
        _proj_heads_kernel,
        grid=(m // block_m, H),
        in_specs=[
            pl.BlockSpec((block_m, D), lambda i, h: (i, 0)),
            pl.BlockSpec((1, D, DH), lambda i, h: (h, 0, 0)),
            pl.BlockSpec((1, 1, DH), lambda i, h: (h, 0, 0)),
        ],
        out_specs=pl.BlockSpec((1, block_m, DH), lambda i, h: (h, i, 0)),
        out_shape=jax.ShapeDtypeStruct((H, m, DH), jnp.float32),
    )(x, w3, b3)


_NCELL = 128


def _scores_kernel(q_ref, k_ref, o_ref, g_ref, *, scale):
    s = (
        lax.dot_general(
            q_ref[0],
            k_ref[0],
            (((1,), (1,)), ((), ())),
            preferred_element_type=jnp.float32,
        )
        * scale
    )
    o_ref[0] = s
    bq, lkv = s.shape
    gm = lax.slice(s, (0, 0), (bq, _NCELL))
    for k in range(1, lkv // _NCELL):
        gm = jnp.maximum(
            gm, lax.slice(s, (0, k * _NCELL), (bq, (k + 1) * _NCELL))
        )
    g_ref[0] = gm


def _scores(q3, k3, B, LQ, LKV, block_q=512):
    H, DH = HEADS, HDIM
    grid = (B, H, LQ // block_q)
    return pl.pallas_call(
        functools.partial(_scores_kernel, scale=1.0 / (DH**0.5)),
        grid=grid,
        in_specs=[
            pl.BlockSpec(
                (1, block_q, DH),
                lambda b, h, i: (h, b * (LQ // block_q) + i, 0),
            ),
            pl.BlockSpec((1, LKV, DH), lambda b, h, i: (h, b, 0)),
        ],
        out_specs=[
            pl.BlockSpec(
                (1, block_q, LKV), lambda b, h, i: (b * HEADS + h, i, 0)
            ),
            pl.BlockSpec(
                (1, block_q, _NCELL), lambda b, h, i: (b * HEADS + h, i, 0)
            ),
        ],
        out_shape=[
            jax.ShapeDtypeStruct((B * H, LQ, LKV), jnp.float32),
            jax.ShapeDtypeStruct((B * H, LQ, _NCELL), jnp.float32),
        ],
    )(q3, k3)


def _vsort_d(x):
    s, _ = plsc.sort_key_val(x, x, descending=True)
    return s


def _rev(x):
    return lax.rev(x, dimensions=(0,))


def _sort64_desc(b0, b1, b2, b3):
    b0, b1, b2, b3 = _vsort_d(b0), _vsort_d(b1), _vsort_d(b2), _vsort_d(b3)

    def merge16(a, c):
        rc = _rev(c)
        return _vsort_d(jnp.maximum(a, rc)), _vsort_d(jnp.minimum(a, rc))

    x0, x1 = merge16(b0, b1)
    y0, y1 = merge16(b2, b3)
    ry0, ry1 = _rev(y1), _rev(y0)
    hi0, hi1 = jnp.maximum(x0, ry0), jnp.maximum(x1, ry1)
    lo0, lo1 = jnp.minimum(x0, ry0), jnp.minimum(x1, ry1)

    def clean32(p0, p1):
        return (
            _vsort_d(jnp.maximum(p0, p1)),
            _vsort_d(jnp.minimum(p0, p1)),
        )

    r0, r1 = clean32(hi0, hi1)
    r2, r3 = clean32(lo0, lo1)
    return r0, r1, r2, r3


def _merge_top64(r0, r1, r2, r3, s0, s1, s2, s3):
    m0 = jnp.maximum(r0, _rev(s3))
    m1 = jnp.maximum(r1, _rev(s2))
    m2 = jnp.maximum(r2, _rev(s1))
    m3 = jnp.maximum(r3, _rev(s0))
    p0, p1 = jnp.maximum(m0, m2), jnp.maximum(m1, m3)
    q0, q1 = jnp.minimum(m0, m2), jnp.minimum(m1, m3)

    def clean32(a, b):
        return (
            _vsort_d(jnp.maximum(a, b)),
            _vsort_d(jnp.minimum(a, b)),
        )

    r0, r1 = clean32(p0, p1)
    r2, r3 = clean32(q0, q1)
    return r0, r1, r2, r3


_SC_CHUNK = 16
_GROUP = 128


def _topk_sc(scores, gmax):
    R, LKV = scores.shape
    n_gmax = gmax.shape[1]
    mesh = plsc.VectorSubcoreMesh(core_axis_name="c", subcore_axis_name="s")
    info = plsc.get_sparse_core_info()
    n_workers = info.num_cores * info.num_subcores
    rows_per_worker = R // n_workers
    n_chunks = rows_per_worker // _SC_CHUNK

    @functools.partial(
        pl.kernel,
        mesh=mesh,
        out_type=jax.ShapeDtypeStruct((R, TOPK), jnp.float32),
        scratch_types=[
            pltpu.VMEM((_SC_CHUNK, LKV), jnp.float32),
            pltpu.VMEM((_SC_CHUNK, TOPK), jnp.float32),
            pltpu.VMEM((_SC_CHUNK, n_gmax), jnp.float32),
            pltpu.VMEM((_SC_CHUNK, LKV + TOPK), jnp.float32),
        ],
        compiler_params=pltpu.CompilerParams(needs_layout_passes=False),
    )
    def topk_kernel(scores_hbm, gmax_hbm, out_hbm, rows_v, out_v, gmax_v, cand_v):
        wid = lax.axis_index("s") * info.num_cores + lax.axis_index("c")
        base = wid * rows_per_worker

        def chunk_body(c, _):
            row0 = base + c * _SC_CHUNK
            pltpu.sync_copy(scores_hbm.at[pl.ds(row0, _SC_CHUNK)], rows_v)
            pltpu.sync_copy(gmax_hbm.at[pl.ds(row0, _SC_CHUNK)], gmax_v)

            @plsc.parallel_loop(0, _SC_CHUNK, 1, unroll=3)
            def row_body(r):
                def ld64(ref, base_el):
                    return (
                        ref[r, pl.ds(base_el, 16)],
                        ref[r, pl.ds(base_el + 16, 16)],
                        ref[r, pl.ds(base_el + 32, 16)],
                        ref[r, pl.ds(base_el + 48, 16)],
                    )

                g0, g1, g2, g3 = ld64(gmax_v, 0)
                r0, r1, r2, r3 = _sort64_desc(g0, g1, g2, g3)

                def t_body(bi, carry):
                    r0, r1, r2, r3 = carry
                    b0, b1, b2, b3 = ld64(gmax_v, bi * 64)
                    s0, s1, s2, s3 = _sort64_desc(b0, b1, b2, b3)
                    return _merge_top64(r0, r1, r2, r3, s0, s1, s2, s3)

                r0, r1, r2, r3 = lax.fori_loop(
                    1, n_gmax // 64, t_body, (r0, r1, r2, r3)
                )
                t0 = jnp.min(r3)

                def c_body(j, cnt):
                    xs, msks, pcs = [], [], []
                    for u in range(8):
                        x = rows_v[r, pl.ds(j * 128 + u * 16, 16)]
                        msk = x >= t0
                        xs.append(x)
                        msks.append(msk)
                        pcs.append(
                            plsc.all_reduce_population_count(msk)[0]
                        )
                    offs = [cnt]
                    for u in range(8):
                        offs.append(offs[-1] + pcs[u])
                    for u in range(8):
                        plsc.store_compressed(
                            cand_v.at[r, pl.ds(offs[u], 16)],
                            xs[u],
                            mask=msks[u],
                        )
                    return offs[-1]

                cnt = lax.fori_loop(0, LKV // 128, c_body, 0)

                neg = jnp.full((16,), -jnp.inf, dtype=jnp.float32)
                cand_v[r, pl.ds(cnt, 16)] = neg
                cand_v[r, pl.ds(cnt + 16, 16)] = neg
                cand_v[r, pl.ds(cnt + 32, 16)] = neg
                cand_v[r, pl.ds(cnt + 48, 16)] = neg

                b0, b1, b2, b3 = ld64(cand_v, 0)
                r0, r1, r2, r3 = _sort64_desc(b0, b1, b2, b3)
                t = jnp.min(r3)
                nb = (cnt + 63) // 64

                def batch_body(bi, carry):
                    r0, r1, r2, r3, t = carry
                    b0, b1, b2, b3 = ld64(cand_v, bi * 64)
                    bmax = jnp.max(
                        jnp.maximum(jnp.maximum(b0, b1), jnp.maximum(b2, b3))
                    )

                    def do_merge(args):
                        r0, r1, r2, r3, b0, b1, b2, b3 = args
                        s0, s1, s2, s3 = _sort64_desc(b0, b1, b2, b3)
                        n0, n1, n2, n3 = _merge_top64(
                            r0, r1, r2, r3, s0, s1, s2, s3
                        )
                        return n0, n1, n2, n3, jnp.min(n3)

                    def no_merge(args):
                        r0, r1, r2, r3, b0, b1, b2, b3 = args
                        return r0, r1, r2, r3, t

                    return lax.cond(
                        bmax > t,
                        do_merge,
                        no_merge,
                        (r0, r1, r2, r3, b0, b1, b2, b3),
                    )

                r0, r1, r2, r3, t = lax.fori_loop(
                    1, nb, batch_body, (r0, r1, r2, r3, t)
                )
                out_v[r, pl.ds(0, 16)] = r0
                out_v[r, pl.ds(16, 16)] = r1
                out_v[r, pl.ds(32, 16)] = r2
                out_v[r, pl.ds(48, 16)] = r3

            pltpu.sync_copy(out_v, out_hbm.at[pl.ds(row0, _SC_CHUNK)])
            return ()

        lax.fori_loop(0, n_chunks, chunk_body, ())

    return topk_kernel(scores, gmax)


def _attend_kernel(q_ref, k_ref, tk_ref, v_ref, att_ref, sm_ref, *, scale):
    tv = tk_ref[0]
    m = tv[:, 0:1]
    e = jnp.exp(tv - m)
    denom = jnp.sum(e, axis=1, keepdims=True)
    thr = tv[:, TOPK - 1 :]
    sm_ref[0] = e / denom
    s = (
        lax.dot_general(
            q_ref[0],
            k_ref[0],
            (((1,), (1,)), ((), ())),
            preferred_element_type=jnp.float32,
        )
        * scale
    )
    p = jnp.where(s >= thr, jnp.exp(s - m), 0.0) / denom
    att_ref[0] = jnp.dot(p, v_ref[0], preferred_element_type=jnp.float32)


def _attend(q3, k3, topk, v3, B, LQ, LKV, block_q=512):
    H, DH = HEADS, HDIM
    BH = B * H
    grid = (BH, LQ // block_q)
    nblk = LQ // block_q
    return pl.pallas_call(
        functools.partial(_attend_kernel, scale=1.0 / (DH**0.5)),
        grid=grid,
        in_specs=[
            pl.BlockSpec(
                (1, block_q, DH),
                lambda bh, i: (bh % HEADS, (bh // HEADS) * nblk + i, 0),
            ),
            pl.BlockSpec(
                (1, LKV, DH), lambda bh, i: (bh % HEADS, bh // HEADS, 0)
            ),
            pl.BlockSpec((1, block_q, TOPK), lambda bh, i: (bh, i, 0)),
            pl.BlockSpec(
                (1, LKV, DH), lambda bh, i: (bh % HEADS, bh // HEADS, 0)
            ),
        ],
        out_specs=[
            pl.BlockSpec((1, block_q, DH), lambda bh, i: (bh, i, 0)),
            pl.BlockSpec((1, block_q, TOPK), lambda bh, i: (bh, i, 0)),
        ],
        out_shape=[
            jax.ShapeDtypeStruct((BH, LQ, DH), jnp.float32),
            jax.ShapeDtypeStruct((BH, LQ, TOPK), jnp.float32),
        ],
    )(q3, k3, topk, v3)


def _out_proj_kernel(a_ref, w_ref, b_ref, o_ref):
    h = pl.program_id(2)

    @pl.when(h == 0)
    def _():
        o_ref[...] = jnp.broadcast_to(b_ref[...], o_ref.shape)

    o_ref[...] += jnp.dot(
        a_ref[0, 0], w_ref[0], preferred_element_type=jnp.float32
    )


def _out_proj(att, Wo, bo, B, LQ, block_m=512):
    H, DH, D = HEADS, HDIM, EMBED
    att4 = att.reshape(B, H, LQ, DH)
    wo3 = Wo.reshape(H, DH, D)
    grid = (B, LQ // block_m, H)
    return pl.pallas_call(
        _out_proj_kernel,
        grid=grid,
        in_specs=[
            pl.BlockSpec((1, 1, block_m, DH), lambda b, i, h: (b, h, i, 0)),
            pl.BlockSpec((1, DH, D), lambda b, i, h: (h, 0, 0)),
            pl.BlockSpec((D,), lambda b, i, h: (0,)),
        ],
        out_specs=pl.BlockSpec(
            (1, block_m, D), lambda b, i, h: (b, i, 0)
        ),
        out_shape=jax.ShapeDtypeStruct((B, LQ, D), jnp.float32),
    )(att4, wo3, bo)


def kernel(local_feat, global_feat, Wq, bq, Wk, bk, Wv, bv, Wo, bo):
    B, LQ, D = local_feat.shape
    LKV = global_feat.shape[1]
    H = HEADS

    q3 = _proj_heads(local_feat.reshape(B * LQ, D), Wq, bq)
    k3 = _proj_heads(global_feat.reshape(B * LKV, D), Wk, bk)
    v3 = _proj_heads(global_feat.reshape(B * LKV, D), Wv, bv)

    scores, gmax = _scores(q3, k3, B, LQ, LKV)

    topk = _topk_sc(
        scores.reshape(B * H * LQ, LKV), gmax.reshape(B * H * LQ, _NCELL)
    ).reshape(B * H, LQ, TOPK)

    att, sm = _attend(q3, k3, topk, v3, B, LQ, LKV)

    output = _out_proj(att, Wo, bo, B, LQ)
    return (output, sm.reshape(B, H, LQ, TOPK))


# speedup vs baseline: 3.7561x; 1.1027x over previous
import functools

import jax
import jax.numpy as jnp
from jax import lax
from jax.experimental import pallas as pl
from jax.experimental.pallas import tpu as pltpu
from jax.experimental.pallas import tpu_sc as plsc

EMBED = 1024
HEADS = 16
HDIM = EMBED // HEADS
TOPK = 64


def _matmul_bias_kernel(x_ref, w_ref, b_ref, o_ref):
    o_ref[...] = (
        jnp.dot(x_ref[...], w_ref[...], preferred_element_type=jnp.float32)
        + b_ref[...]
    )


def _matmul_bias(x, w, b, block_m=512):
    m, kdim = x.shape
    n = w.shape[1]
    return pl.pallas_call(
        _matmul_bias_kernel,
        grid=(m // block_m,),
        in_specs=[
            pl.BlockSpec((block_m, kdim), lambda i: (i, 0)),
            pl.BlockSpec((kdim, n), lambda i: (0, 0)),
            pl.BlockSpec((n,), lambda i: (0,)),
        ],
        out_specs=pl.BlockSpec((block_m, n), lambda i: (i, 0)),
        out_shape=jax.ShapeDtypeStruct((m, n), jnp.float32),
    )(x, w, b)


def _proj_heads_kernel(x_ref, w_ref, b_ref, o_ref):
    o_ref[0] = (
        jnp.dot(x_ref[...], w_ref[0], preferred_element_type=jnp.float32)
        + b_ref[0]
    )


def _proj_heads(x, w, b, block_m=512):
    m, D = x.shape
    H, DH = HEADS, HDIM
    w3 = w.reshape(D, H, DH).transpose(1, 0, 2)
    b3 = b.reshape(H, 1, DH)
    return pl.pallas_call(
        _proj_heads_kernel,
        grid=(m // block_m, H),
        in_specs=[
            pl.BlockSpec((block_m, D), lambda i, h: (i, 0)),
            pl.BlockSpec((1, D, DH), lambda i, h: (h, 0, 0)),
            pl.BlockSpec((1, 1, DH), lambda i, h: (h, 0, 0)),
        ],
        out_specs=pl.BlockSpec((1, block_m, DH), lambda i, h: (h, i, 0)),
        out_shape=jax.ShapeDtypeStruct((H, m, DH), jnp.float32),
    )(x, w3, b3)


_NCELL = 128


def _scores_kernel(q_ref, k_ref, o_ref, g_ref, *, scale):
    s = (
        lax.dot_general(
            q_ref[0],
            k_ref[0],
            (((1,), (1,)), ((), ())),
            preferred_element_type=jnp.float32,
        )
        * scale
    )
    o_ref[0] = s
    bq, lkv = s.shape
    gm = lax.slice(s, (0, 0), (bq, _NCELL))
    for k in range(1, lkv // _NCELL):
        gm = jnp.maximum(
            gm, lax.slice(s, (0, k * _NCELL), (bq, (k + 1) * _NCELL))
        )
    g_ref[0] = gm


def _scores(q3, k3, B, LQ, LKV, block_q=512):
    H, DH = HEADS, HDIM
    grid = (B, H, LQ // block_q)
    return pl.pallas_call(
        functools.partial(_scores_kernel, scale=1.0 / (DH**0.5)),
        grid=grid,
        in_specs=[
            pl.BlockSpec(
                (1, block_q, DH),
                lambda b, h, i: (h, b * (LQ // block_q) + i, 0),
            ),
            pl.BlockSpec((1, LKV, DH), lambda b, h, i: (h, b, 0)),
        ],
        out_specs=[
            pl.BlockSpec(
                (1, block_q, LKV), lambda b, h, i: (b * HEADS + h, i, 0)
            ),
            pl.BlockSpec(
                (1, block_q, _NCELL), lambda b, h, i: (b * HEADS + h, i, 0)
            ),
        ],
        out_shape=[
            jax.ShapeDtypeStruct((B * H, LQ, LKV), jnp.float32),
            jax.ShapeDtypeStruct((B * H, LQ, _NCELL), jnp.float32),
        ],
    )(q3, k3)


def _vsort_d(x):
    s, _ = plsc.sort_key_val(x, x, descending=True)
    return s


def _rev(x):
    return lax.rev(x, dimensions=(0,))


def _sort64_desc(b0, b1, b2, b3):
    b0, b1, b2, b3 = _vsort_d(b0), _vsort_d(b1), _vsort_d(b2), _vsort_d(b3)

    def merge16(a, c):
        rc = _rev(c)
        return _vsort_d(jnp.maximum(a, rc)), _vsort_d(jnp.minimum(a, rc))

    x0, x1 = merge16(b0, b1)
    y0, y1 = merge16(b2, b3)
    ry0, ry1 = _rev(y1), _rev(y0)
    hi0, hi1 = jnp.maximum(x0, ry0), jnp.maximum(x1, ry1)
    lo0, lo1 = jnp.minimum(x0, ry0), jnp.minimum(x1, ry1)

    def clean32(p0, p1):
        return (
            _vsort_d(jnp.maximum(p0, p1)),
            _vsort_d(jnp.minimum(p0, p1)),
        )

    r0, r1 = clean32(hi0, hi1)
    r2, r3 = clean32(lo0, lo1)
    return r0, r1, r2, r3


def _merge_top64(r0, r1, r2, r3, s0, s1, s2, s3):
    m0 = jnp.maximum(r0, _rev(s3))
    m1 = jnp.maximum(r1, _rev(s2))
    m2 = jnp.maximum(r2, _rev(s1))
    m3 = jnp.maximum(r3, _rev(s0))
    p0, p1 = jnp.maximum(m0, m2), jnp.maximum(m1, m3)
    q0, q1 = jnp.minimum(m0, m2), jnp.minimum(m1, m3)

    def clean32(a, b):
        return (
            _vsort_d(jnp.maximum(a, b)),
            _vsort_d(jnp.minimum(a, b)),
        )

    r0, r1 = clean32(p0, p1)
    r2, r3 = clean32(q0, q1)
    return r0, r1, r2, r3


_SC_CHUNK = 16
_GROUP = 128


def _topk_sc(scores, gmax):
    R, LKV = scores.shape
    n_gmax = gmax.shape[1]
    mesh = plsc.VectorSubcoreMesh(core_axis_name="c", subcore_axis_name="s")
    info = plsc.get_sparse_core_info()
    n_workers = info.num_cores * info.num_subcores
    rows_per_worker = R // n_workers
    n_chunks = rows_per_worker // _SC_CHUNK

    @functools.partial(
        pl.kernel,
        mesh=mesh,
        out_type=jax.ShapeDtypeStruct((R, TOPK), jnp.float32),
        scratch_types=[
            pltpu.VMEM((2, _SC_CHUNK, LKV), jnp.float32),
            pltpu.VMEM((_SC_CHUNK, TOPK), jnp.float32),
            pltpu.VMEM((_SC_CHUNK, n_gmax), jnp.float32),
            pltpu.VMEM((_SC_CHUNK, LKV + TOPK), jnp.float32),
            pltpu.SemaphoreType.DMA((2,)),
        ],
        compiler_params=pltpu.CompilerParams(needs_layout_passes=False),
    )
    def topk_kernel(
        scores_hbm, gmax_hbm, out_hbm, rows2_v, out_v, gmax_v, cand_v, sems
    ):
        wid = lax.axis_index("s") * info.num_cores + lax.axis_index("c")
        base = wid * rows_per_worker

        pltpu.async_copy(
            scores_hbm.at[pl.ds(base, _SC_CHUNK)], rows2_v.at[0], sems.at[0]
        )

        def chunk_body(c, _):
            buf = lax.rem(c, 2)
            row0 = base + c * _SC_CHUNK

            @pl.when(c + 1 < n_chunks)
            def _():
                pltpu.async_copy(
                    scores_hbm.at[pl.ds(row0 + _SC_CHUNK, _SC_CHUNK)],
                    rows2_v.at[1 - buf],
                    sems.at[1 - buf],
                )

            pltpu.sync_copy(gmax_hbm.at[pl.ds(row0, _SC_CHUNK)], gmax_v)
            pltpu.make_async_copy(
                scores_hbm.at[pl.ds(row0, _SC_CHUNK)],
                rows2_v.at[buf],
                sems.at[buf],
            ).wait()

            @plsc.parallel_loop(0, _SC_CHUNK, 1, unroll=3)
            def row_body(r):
                def ld64(ref, base_el):
                    return (
                        ref[r, pl.ds(base_el, 16)],
                        ref[r, pl.ds(base_el + 16, 16)],
                        ref[r, pl.ds(base_el + 32, 16)],
                        ref[r, pl.ds(base_el + 48, 16)],
                    )

                g0, g1, g2, g3 = ld64(gmax_v, 0)
                r0, r1, r2, r3 = _sort64_desc(g0, g1, g2, g3)

                def t_body(bi, carry):
                    r0, r1, r2, r3 = carry
                    b0, b1, b2, b3 = ld64(gmax_v, bi * 64)
                    s0, s1, s2, s3 = _sort64_desc(b0, b1, b2, b3)
                    return _merge_top64(r0, r1, r2, r3, s0, s1, s2, s3)

                r0, r1, r2, r3 = lax.fori_loop(
                    1, n_gmax // 64, t_body, (r0, r1, r2, r3)
                )
                t0 = jnp.min(r3)

                def c_body(j, cnt):
                    xs, msks, pcs = [], [], []
                    for u in range(8):
                        x = rows2_v[buf, r, pl.ds(j * 128 + u * 16, 16)]
                        msk = x >= t0
                        xs.append(x)
                        msks.append(msk)
                        pcs.append(
                            plsc.all_reduce_population_count(msk)[0]
                        )
                    offs = [cnt]
                    for u in range(8):
                        offs.append(offs[-1] + pcs[u])
                    for u in range(8):
                        plsc.store_compressed(
                            cand_v.at[r, pl.ds(offs[u], 16)],
                            xs[u],
                            mask=msks[u],
                        )
                    return offs[-1]

                cnt = lax.fori_loop(0, LKV // 128, c_body, 0)

                neg = jnp.full((16,), -jnp.inf, dtype=jnp.float32)
                cand_v[r, pl.ds(cnt, 16)] = neg
                cand_v[r, pl.ds(cnt + 16, 16)] = neg
                cand_v[r, pl.ds(cnt + 32, 16)] = neg
                cand_v[r, pl.ds(cnt + 48, 16)] = neg

                b0, b1, b2, b3 = ld64(cand_v, 0)
                r0, r1, r2, r3 = _sort64_desc(b0, b1, b2, b3)
                t = jnp.min(r3)
                nb = (cnt + 63) // 64

                def batch_body(bi, carry):
                    r0, r1, r2, r3, t = carry
                    b0, b1, b2, b3 = ld64(cand_v, bi * 64)
                    bmax = jnp.max(
                        jnp.maximum(jnp.maximum(b0, b1), jnp.maximum(b2, b3))
                    )

                    def do_merge(args):
                        r0, r1, r2, r3, b0, b1, b2, b3 = args
                        s0, s1, s2, s3 = _sort64_desc(b0, b1, b2, b3)
                        n0, n1, n2, n3 = _merge_top64(
                            r0, r1, r2, r3, s0, s1, s2, s3
                        )
                        return n0, n1, n2, n3, jnp.min(n3)

                    def no_merge(args):
                        r0, r1, r2, r3, b0, b1, b2, b3 = args
                        return r0, r1, r2, r3, t

                    return lax.cond(
                        bmax > t,
                        do_merge,
                        no_merge,
                        (r0, r1, r2, r3, b0, b1, b2, b3),
                    )

                r0, r1, r2, r3, t = lax.fori_loop(
                    1, nb, batch_body, (r0, r1, r2, r3, t)
                )
                out_v[r, pl.ds(0, 16)] = r0
                out_v[r, pl.ds(16, 16)] = r1
                out_v[r, pl.ds(32, 16)] = r2
                out_v[r, pl.ds(48, 16)] = r3

            pltpu.sync_copy(out_v, out_hbm.at[pl.ds(row0, _SC_CHUNK)])
            return ()

        lax.fori_loop(0, n_chunks, chunk_body, ())

    return topk_kernel(scores, gmax)


def _attend_kernel(q_ref, k_ref, tk_ref, v_ref, att_ref, sm_ref, *, scale):
    tv = tk_ref[0]
    m = tv[:, 0:1]
    e = jnp.exp(tv - m)
    denom = jnp.sum(e, axis=1, keepdims=True)
    thr = tv[:, TOPK - 1 :]
    sm_ref[0] = e / denom
    s = (
        lax.dot_general(
            q_ref[0],
            k_ref[0],
            (((1,), (1,)), ((), ())),
            preferred_element_type=jnp.float32,
        )
        * scale
    )
    p = jnp.where(s >= thr, jnp.exp(s - m), 0.0) / denom
    att_ref[0] = jnp.dot(p, v_ref[0], preferred_element_type=jnp.float32)


def _attend(q3, k3, topk, v3, B, LQ, LKV, block_q=512):
    H, DH = HEADS, HDIM
    BH = B * H
    grid = (BH, LQ // block_q)
    nblk = LQ // block_q
    return pl.pallas_call(
        functools.partial(_attend_kernel, scale=1.0 / (DH**0.5)),
        grid=grid,
        in_specs=[
            pl.BlockSpec(
                (1, block_q, DH),
                lambda bh, i: (bh % HEADS, (bh // HEADS) * nblk + i, 0),
            ),
            pl.BlockSpec(
                (1, LKV, DH), lambda bh, i: (bh % HEADS, bh // HEADS, 0)
            ),
            pl.BlockSpec((1, block_q, TOPK), lambda bh, i: (bh, i, 0)),
            pl.BlockSpec(
                (1, LKV, DH), lambda bh, i: (bh % HEADS, bh // HEADS, 0)
            ),
        ],
        out_specs=[
            pl.BlockSpec((1, block_q, DH), lambda bh, i: (bh, i, 0)),
            pl.BlockSpec((1, block_q, TOPK), lambda bh, i: (bh, i, 0)),
        ],
        out_shape=[
            jax.ShapeDtypeStruct((BH, LQ, DH), jnp.float32),
            jax.ShapeDtypeStruct((BH, LQ, TOPK), jnp.float32),
        ],
    )(q3, k3, topk, v3)


def _out_proj_kernel(a_ref, w_ref, b_ref, o_ref):
    h = pl.program_id(2)

    @pl.when(h == 0)
    def _():
        o_ref[...] = jnp.broadcast_to(b_ref[...], o_ref.shape)

    o_ref[...] += jnp.dot(
        a_ref[0, 0], w_ref[0], preferred_element_type=jnp.float32
    )


def _out_proj(att, Wo, bo, B, LQ, block_m=512):
    H, DH, D = HEADS, HDIM, EMBED
    att4 = att.reshape(B, H, LQ, DH)
    wo3 = Wo.reshape(H, DH, D)
    grid = (B, LQ // block_m, H)
    return pl.pallas_call(
        _out_proj_kernel,
        grid=grid,
        in_specs=[
            pl.BlockSpec((1, 1, block_m, DH), lambda b, i, h: (b, h, i, 0)),
            pl.BlockSpec((1, DH, D), lambda b, i, h: (h, 0, 0)),
            pl.BlockSpec((D,), lambda b, i, h: (0,)),
        ],
        out_specs=pl.BlockSpec(
            (1, block_m, D), lambda b, i, h: (b, i, 0)
        ),
        out_shape=jax.ShapeDtypeStruct((B, LQ, D), jnp.float32),
    )(att4, wo3, bo)


def kernel(local_feat, global_feat, Wq, bq, Wk, bk, Wv, bv, Wo, bo):
    B, LQ, D = local_feat.shape
    LKV = global_feat.shape[1]
    H = HEADS

    q3 = _proj_heads(local_feat.reshape(B * LQ, D), Wq, bq)
    k3 = _proj_heads(global_feat.reshape(B * LKV, D), Wk, bk)
    v3 = _proj_heads(global_feat.reshape(B * LKV, D), Wv, bv)

    scores, gmax = _scores(q3, k3, B, LQ, LKV)

    topk = _topk_sc(
        scores.reshape(B * H * LQ, LKV), gmax.reshape(B * H * LQ, _NCELL)
    ).reshape(B * H, LQ, TOPK)

    att, sm = _attend(q3, k3, topk, v3, B, LQ, LKV)

    output = _out_proj(att, Wo, bo, B, LQ)
    return (output, sm.reshape(B, H, LQ, TOPK))


# speedup vs baseline: 3.7567x; 1.0001x over previous
import functools

import jax
import jax.numpy as jnp
from jax import lax
from jax.experimental import pallas as pl
from jax.experimental.pallas import tpu as pltpu
from jax.experimental.pallas import tpu_sc as plsc

EMBED = 1024
HEADS = 16
HDIM = EMBED // HEADS
TOPK = 64


def _proj_heads_kernel(x_ref, w_ref, b_ref, o_ref):
    o_ref[0] = (
        jnp.dot(x_ref[...], w_ref[0], preferred_element_type=jnp.float32)
        + b_ref[0]
    )


def _proj_heads(x, w, b, block_m=512):
    m, D = x.shape
    H, DH = HEADS, HDIM
    w3 = w.reshape(D, H, DH).transpose(1, 0, 2)
    b3 = b.reshape(H, 1, DH)
    return pl.pallas_call(
        _proj_heads_kernel,
        grid=(m // block_m, H),
        in_specs=[
            pl.BlockSpec((block_m, D), lambda i, h: (i, 0)),
            pl.BlockSpec((1, D, DH), lambda i, h: (h, 0, 0)),
            pl.BlockSpec((1, 1, DH), lambda i, h: (h, 0, 0)),
        ],
        out_specs=pl.BlockSpec((1, block_m, DH), lambda i, h: (h, i, 0)),
        out_shape=jax.ShapeDtypeStruct((H, m, DH), jnp.float32),
    )(x, w3, b3)


_NCELL = 128


def _scores_kernel(q_ref, k_ref, o_ref, g_ref, *, scale):
    s = (
        lax.dot_general(
            q_ref[0],
            k_ref[0],
            (((1,), (1,)), ((), ())),
            preferred_element_type=jnp.float32,
        )
        * scale
    )
    o_ref[0] = s
    bq, lkv = s.shape
    gm = lax.slice(s, (0, 0), (bq, _NCELL))
    for k in range(1, lkv // _NCELL):
        gm = jnp.maximum(
            gm, lax.slice(s, (0, k * _NCELL), (bq, (k + 1) * _NCELL))
        )
    g_ref[0] = gm


def _scores(q3, k3, B, LQ, LKV, block_q=512):
    H, DH = HEADS, HDIM
    grid = (B, H, LQ // block_q)
    return pl.pallas_call(
        functools.partial(_scores_kernel, scale=1.0 / (DH**0.5)),
        grid=grid,
        in_specs=[
            pl.BlockSpec(
                (1, block_q, DH),
                lambda b, h, i: (h, b * (LQ // block_q) + i, 0),
            ),
            pl.BlockSpec((1, LKV, DH), lambda b, h, i: (h, b, 0)),
        ],
        out_specs=[
            pl.BlockSpec(
                (1, block_q, LKV), lambda b, h, i: (b * HEADS + h, i, 0)
            ),
            pl.BlockSpec(
                (1, block_q, _NCELL), lambda b, h, i: (b * HEADS + h, i, 0)
            ),
        ],
        out_shape=[
            jax.ShapeDtypeStruct((B * H, LQ, LKV), jnp.float32),
            jax.ShapeDtypeStruct((B * H, LQ, _NCELL), jnp.float32),
        ],
    )(q3, k3)


def _vsort_d(x):
    s, _ = plsc.sort_key_val(x, x, descending=True)
    return s


def _rev(x):
    return lax.rev(x, dimensions=(0,))


def _sort64_desc(b0, b1, b2, b3):
    b0, b1, b2, b3 = _vsort_d(b0), _vsort_d(b1), _vsort_d(b2), _vsort_d(b3)

    def merge16(a, c):
        rc = _rev(c)
        return _vsort_d(jnp.maximum(a, rc)), _vsort_d(jnp.minimum(a, rc))

    x0, x1 = merge16(b0, b1)
    y0, y1 = merge16(b2, b3)
    ry0, ry1 = _rev(y1), _rev(y0)
    hi0, hi1 = jnp.maximum(x0, ry0), jnp.maximum(x1, ry1)
    lo0, lo1 = jnp.minimum(x0, ry0), jnp.minimum(x1, ry1)

    def clean32(p0, p1):
        return (
            _vsort_d(jnp.maximum(p0, p1)),
            _vsort_d(jnp.minimum(p0, p1)),
        )

    r0, r1 = clean32(hi0, hi1)
    r2, r3 = clean32(lo0, lo1)
    return r0, r1, r2, r3


def _merge_top64(r0, r1, r2, r3, s0, s1, s2, s3):
    m0 = jnp.maximum(r0, _rev(s3))
    m1 = jnp.maximum(r1, _rev(s2))
    m2 = jnp.maximum(r2, _rev(s1))
    m3 = jnp.maximum(r3, _rev(s0))
    p0, p1 = jnp.maximum(m0, m2), jnp.maximum(m1, m3)
    q0, q1 = jnp.minimum(m0, m2), jnp.minimum(m1, m3)

    def clean32(a, b):
        return (
            _vsort_d(jnp.maximum(a, b)),
            _vsort_d(jnp.minimum(a, b)),
        )

    r0, r1 = clean32(p0, p1)
    r2, r3 = clean32(q0, q1)
    return r0, r1, r2, r3


_SC_CHUNK = 16
def _topk_sc(scores, gmax):
    R, LKV = scores.shape
    n_gmax = gmax.shape[1]
    mesh = plsc.VectorSubcoreMesh(core_axis_name="c", subcore_axis_name="s")
    info = plsc.get_sparse_core_info()
    n_workers = info.num_cores * info.num_subcores
    rows_per_worker = R // n_workers
    n_chunks = rows_per_worker // _SC_CHUNK

    @functools.partial(
        pl.kernel,
        mesh=mesh,
        out_type=jax.ShapeDtypeStruct((R, TOPK), jnp.float32),
        scratch_types=[
            pltpu.VMEM((2, _SC_CHUNK, LKV), jnp.float32),
            pltpu.VMEM((_SC_CHUNK, TOPK), jnp.float32),
            pltpu.VMEM((_SC_CHUNK, n_gmax), jnp.float32),
            pltpu.VMEM((_SC_CHUNK, LKV + TOPK), jnp.float32),
            pltpu.SemaphoreType.DMA((2,)),
        ],
        compiler_params=pltpu.CompilerParams(needs_layout_passes=False),
    )
    def topk_kernel(
        scores_hbm, gmax_hbm, out_hbm, rows2_v, out_v, gmax_v, cand_v, sems
    ):
        wid = lax.axis_index("s") * info.num_cores + lax.axis_index("c")
        base = wid * rows_per_worker

        pltpu.async_copy(
            scores_hbm.at[pl.ds(base, _SC_CHUNK)], rows2_v.at[0], sems.at[0]
        )

        def chunk_body(c, _):
            buf = lax.rem(c, 2)
            row0 = base + c * _SC_CHUNK

            @pl.when(c + 1 < n_chunks)
            def _():
                pltpu.async_copy(
                    scores_hbm.at[pl.ds(row0 + _SC_CHUNK, _SC_CHUNK)],
                    rows2_v.at[1 - buf],
                    sems.at[1 - buf],
                )

            pltpu.sync_copy(gmax_hbm.at[pl.ds(row0, _SC_CHUNK)], gmax_v)
            pltpu.make_async_copy(
                scores_hbm.at[pl.ds(row0, _SC_CHUNK)],
                rows2_v.at[buf],
                sems.at[buf],
            ).wait()

            @plsc.parallel_loop(0, _SC_CHUNK, 1, unroll=3)
            def row_body(r):
                def ld64(ref, base_el):
                    return (
                        ref[r, pl.ds(base_el, 16)],
                        ref[r, pl.ds(base_el + 16, 16)],
                        ref[r, pl.ds(base_el + 32, 16)],
                        ref[r, pl.ds(base_el + 48, 16)],
                    )

                g0, g1, g2, g3 = ld64(gmax_v, 0)
                r0, r1, r2, r3 = _sort64_desc(g0, g1, g2, g3)

                def t_body(bi, carry):
                    r0, r1, r2, r3 = carry
                    b0, b1, b2, b3 = ld64(gmax_v, bi * 64)
                    s0, s1, s2, s3 = _sort64_desc(b0, b1, b2, b3)
                    return _merge_top64(r0, r1, r2, r3, s0, s1, s2, s3)

                r0, r1, r2, r3 = lax.fori_loop(
                    1, n_gmax // 64, t_body, (r0, r1, r2, r3)
                )
                t0 = jnp.min(r3)

                def c_body(j, cnt):
                    xs, msks, pcs = [], [], []
                    for u in range(8):
                        x = rows2_v[buf, r, pl.ds(j * 128 + u * 16, 16)]
                        msk = x >= t0
                        xs.append(x)
                        msks.append(msk)
                        pcs.append(
                            plsc.all_reduce_population_count(msk)[0]
                        )
                    offs = [cnt]
                    for u in range(8):
                        offs.append(offs[-1] + pcs[u])
                    for u in range(8):
                        plsc.store_compressed(
                            cand_v.at[r, pl.ds(offs[u], 16)],
                            xs[u],
                            mask=msks[u],
                        )
                    return offs[-1]

                cnt = lax.fori_loop(0, LKV // 128, c_body, 0)

                neg = jnp.full((16,), -jnp.inf, dtype=jnp.float32)
                cand_v[r, pl.ds(cnt, 16)] = neg
                cand_v[r, pl.ds(cnt + 16, 16)] = neg
                cand_v[r, pl.ds(cnt + 32, 16)] = neg
                cand_v[r, pl.ds(cnt + 48, 16)] = neg

                b0, b1, b2, b3 = ld64(cand_v, 0)
                r0, r1, r2, r3 = _sort64_desc(b0, b1, b2, b3)
                t = jnp.min(r3)
                nb = (cnt + 63) // 64

                def batch_body(bi, carry):
                    r0, r1, r2, r3, t = carry
                    b0, b1, b2, b3 = ld64(cand_v, bi * 64)
                    bmax = jnp.max(
                        jnp.maximum(jnp.maximum(b0, b1), jnp.maximum(b2, b3))
                    )

                    def do_merge(args):
                        r0, r1, r2, r3, b0, b1, b2, b3 = args
                        s0, s1, s2, s3 = _sort64_desc(b0, b1, b2, b3)
                        n0, n1, n2, n3 = _merge_top64(
                            r0, r1, r2, r3, s0, s1, s2, s3
                        )
                        return n0, n1, n2, n3, jnp.min(n3)

                    def no_merge(args):
                        r0, r1, r2, r3, b0, b1, b2, b3 = args
                        return r0, r1, r2, r3, t

                    return lax.cond(
                        bmax > t,
                        do_merge,
                        no_merge,
                        (r0, r1, r2, r3, b0, b1, b2, b3),
                    )

                r0, r1, r2, r3, t = lax.fori_loop(
                    1, nb, batch_body, (r0, r1, r2, r3, t)
                )
                out_v[r, pl.ds(0, 16)] = r0
                out_v[r, pl.ds(16, 16)] = r1
                out_v[r, pl.ds(32, 16)] = r2
                out_v[r, pl.ds(48, 16)] = r3

            pltpu.sync_copy(out_v, out_hbm.at[pl.ds(row0, _SC_CHUNK)])
            return ()

        lax.fori_loop(0, n_chunks, chunk_body, ())

    return topk_kernel(scores, gmax)


def _attend_kernel(q_ref, k_ref, tk_ref, v_ref, att_ref, sm_ref, *, scale):
    tv = tk_ref[0]
    m = tv[:, 0:1]
    e = jnp.exp(tv - m)
    denom = jnp.sum(e, axis=1, keepdims=True)
    thr = tv[:, TOPK - 1 :]
    sm_ref[0] = e / denom
    s = (
        lax.dot_general(
            q_ref[0],
            k_ref[0],
            (((1,), (1,)), ((), ())),
            preferred_element_type=jnp.float32,
        )
        * scale
    )
    p = jnp.where(s >= thr, jnp.exp(s - m), 0.0) / denom
    att_ref[0] = jnp.dot(p, v_ref[0], preferred_element_type=jnp.float32)


def _attend(q3, k3, topk, v3, B, LQ, LKV, block_q=512):
    H, DH = HEADS, HDIM
    BH = B * H
    grid = (BH, LQ // block_q)
    nblk = LQ // block_q
    return pl.pallas_call(
        functools.partial(_attend_kernel, scale=1.0 / (DH**0.5)),
        grid=grid,
        in_specs=[
            pl.BlockSpec(
                (1, block_q, DH),
                lambda bh, i: (bh % HEADS, (bh // HEADS) * nblk + i, 0),
            ),
            pl.BlockSpec(
                (1, LKV, DH), lambda bh, i: (bh % HEADS, bh // HEADS, 0)
            ),
            pl.BlockSpec((1, block_q, TOPK), lambda bh, i: (bh, i, 0)),
            pl.BlockSpec(
                (1, LKV, DH), lambda bh, i: (bh % HEADS, bh // HEADS, 0)
            ),
        ],
        out_specs=[
            pl.BlockSpec((1, block_q, DH), lambda bh, i: (bh, i, 0)),
            pl.BlockSpec((1, block_q, TOPK), lambda bh, i: (bh, i, 0)),
        ],
        out_shape=[
            jax.ShapeDtypeStruct((BH, LQ, DH), jnp.float32),
            jax.ShapeDtypeStruct((BH, LQ, TOPK), jnp.float32),
        ],
    )(q3, k3, topk, v3)


def _out_proj_kernel(a_ref, w_ref, b_ref, o_ref):
    h = pl.program_id(2)

    @pl.when(h == 0)
    def _():
        o_ref[...] = jnp.broadcast_to(b_ref[...], o_ref.shape)

    o_ref[...] += jnp.dot(
        a_ref[0, 0], w_ref[0], preferred_element_type=jnp.float32
    )


def _out_proj(att, Wo, bo, B, LQ, block_m=512):
    H, DH, D = HEADS, HDIM, EMBED
    att4 = att.reshape(B, H, LQ, DH)
    wo3 = Wo.reshape(H, DH, D)
    grid = (B, LQ // block_m, H)
    return pl.pallas_call(
        _out_proj_kernel,
        grid=grid,
        in_specs=[
            pl.BlockSpec((1, 1, block_m, DH), lambda b, i, h: (b, h, i, 0)),
            pl.BlockSpec((1, DH, D), lambda b, i, h: (h, 0, 0)),
            pl.BlockSpec((D,), lambda b, i, h: (0,)),
        ],
        out_specs=pl.BlockSpec(
            (1, block_m, D), lambda b, i, h: (b, i, 0)
        ),
        out_shape=jax.ShapeDtypeStruct((B, LQ, D), jnp.float32),
    )(att4, wo3, bo)


def kernel(local_feat, global_feat, Wq, bq, Wk, bk, Wv, bv, Wo, bo):
    B, LQ, D = local_feat.shape
    LKV = global_feat.shape[1]
    H = HEADS

    q3 = _proj_heads(local_feat.reshape(B * LQ, D), Wq, bq)
    k3 = _proj_heads(global_feat.reshape(B * LKV, D), Wk, bk)
    v3 = _proj_heads(global_feat.reshape(B * LKV, D), Wv, bv)

    scores, gmax = _scores(q3, k3, B, LQ, LKV)

    topk = _topk_sc(
        scores.reshape(B * H * LQ, LKV), gmax.reshape(B * H * LQ, _NCELL)
    ).reshape(B * H, LQ, TOPK)

    att, sm = _attend(q3, k3, topk, v3, B, LQ, LKV)

    output = _out_proj(att, Wo, bo, B, LQ)
    return (output, sm.reshape(B, H, LQ, TOPK))
